# baseline (speedup 1.0000x reference)
import jax
import jax.numpy as jnp
from jax import lax
from jax.experimental import pallas as pl
from jax.experimental.pallas import tpu as pltpu
from jax.experimental.pallas import tpu_sc as plsc

B = 16384
HIST = 50
D = 200
DP = 208
H = 256
NW = 32
BPW = B // NW
OCH = 64
NCH = 128
LANES = 16
NFULL = D // LANES


def _sc_body(src_hbm, node_hbm, emb_hbm, se_hbm, ne_hbm,
             idx_all, buf0, buf1, out_v, idxn, nbuf, sem0, sem1, semn):
    wid = lax.axis_index("s") * 2 + lax.axis_index("c")
    wbase = pl.multiple_of(wid * BPW, BPW)

    pltpu.sync_copy(src_hbm.at[pl.ds(wbase, BPW), :], idx_all)

    pltpu.sync_copy(node_hbm.at[pl.ds(wbase, BPW)], idxn)

    def node_chunk(g, c):
        off = pl.multiple_of(g * NCH, NCH)
        pltpu.async_copy(emb_hbm.at[idxn.at[pl.ds(off, NCH)]], nbuf, semn).wait()
        pltpu.sync_copy(nbuf, ne_hbm.at[pl.ds(pl.multiple_of(wbase + off, NCH), NCH), :])
        return c

    lax.fori_loop(0, BPW // NCH, node_chunk, 0)

    zeros16 = jnp.zeros((LANES,), jnp.float32)

    def zinit(r, c):
        out_v[r, pl.ds(NFULL * LANES, LANES)] = zeros16
        return c

    lax.fori_loop(0, OCH, zinit, 0)

    bufs = (buf0, buf1)
    sems = (sem0, sem1)

    def start(s, k):
        pltpu.async_copy(emb_hbm.at[idx_all.at[s]], bufs[k], sems[k])

    def wait(s, k):
        pltpu.make_async_copy(emb_hbm.at[idx_all.at[s]], bufs[k], sems[k]).wait()

    def reduce_row(s, k):
        buf = bufs[k]

        def red(j, accs):
            new = [accs[c] + buf[j, pl.ds(c * LANES, LANES)] for c in range(NFULL)]
            new.append(accs[NFULL] + buf[j, pl.ds(D - LANES, LANES)])
            return tuple(new)

        accs = lax.fori_loop(
            0, HIST, red, tuple(jnp.zeros((LANES,), jnp.float32) for _ in range(NFULL + 1))
        )
        scale = jnp.float32(1.0 / HIST)
        r = lax.rem(s, OCH)
        for c in range(NFULL):
            out_v[r, pl.ds(c * LANES, LANES)] = accs[c] * scale
        out_v[r, pl.ds(D - LANES, LANES)] = accs[NFULL] * scale

    start(0, 0)
    start(1, 1)

    def step(g, c):
        s0 = 2 * g
        s1 = s0 + 1
        wait(s0, 0)
        reduce_row(s0, 0)

        @pl.when(s0 + 2 < BPW)
        def _():
            start(s0 + 2, 0)

        wait(s1, 1)
        reduce_row(s1, 1)

        @pl.when(s1 + 2 < BPW)
        def _():
            start(s1 + 2, 1)

        @pl.when(lax.rem(s1, OCH) == OCH - 1)
        def _():
            off = pl.multiple_of(wbase + s1 - (OCH - 1), OCH)
            pltpu.sync_copy(out_v, se_hbm.at[pl.ds(off, OCH), :])

        return c

    lax.fori_loop(0, BPW // 2, step, 0)


def _sc_gather_mean(source, node, emb):
    mesh = plsc.VectorSubcoreMesh(core_axis_name="c", subcore_axis_name="s")
    f32 = jnp.float32
    run = pl.kernel(
        _sc_body,
        out_type=(
            jax.ShapeDtypeStruct((B, DP), f32),
            jax.ShapeDtypeStruct((B, D), f32),
        ),
        mesh=mesh,
        scratch_types=[
            pltpu.VMEM((BPW, HIST), jnp.int32),
            pltpu.VMEM((HIST, D), f32),
            pltpu.VMEM((HIST, D), f32),
            pltpu.VMEM((OCH, DP), f32),
            pltpu.VMEM((BPW,), jnp.int32),
            pltpu.VMEM((NCH, D), f32),
            pltpu.SemaphoreType.DMA,
            pltpu.SemaphoreType.DMA,
            pltpu.SemaphoreType.DMA,
        ],
        compiler_params=pltpu.CompilerParams(use_tc_tiling_on_sc=False),
    )
    return run(source, node, emb)


NN = 1000000
TBLK = 2048


def _transpose_body(src, dst):
    dst[...] = src[...].T


def _transpose_table(embT):
    return pl.pallas_call(
        _transpose_body,
        grid=(pl.cdiv(NN, TBLK),),
        in_specs=[pl.BlockSpec((D, TBLK), lambda i: (0, i))],
        out_specs=pl.BlockSpec((TBLK, D), lambda i: (i, 0)),
        out_shape=jax.ShapeDtypeStruct((NN, D), jnp.float32),
    )(embT)


BBLK = 2048
NBLK = B // BBLK


def _mlp1_body(se, ne, w1a, w1b, b1, x1, ps, psq):
    x = jnp.dot(se[...], w1a[...], preferred_element_type=jnp.float32)
    x = x + jnp.dot(ne[...], w1b[...], preferred_element_type=jnp.float32)
    x = jnp.maximum(x + b1[...], 0.0)
    x1[...] = x
    ps[...] = jnp.sum(x, axis=0, keepdims=True).reshape(1, 1, H)
    psq[...] = jnp.sum(x * x, axis=0, keepdims=True).reshape(1, 1, H)


def _mlp2_body(x1, ps, psq, g1, be1, w2, b2, y, ps2, psq2):
    m = jnp.sum(ps[...].reshape(NBLK, H), axis=0, keepdims=True) * (1.0 / B)
    ex2 = jnp.sum(psq[...].reshape(NBLK, H), axis=0, keepdims=True) * (1.0 / B)
    inv = lax.rsqrt(ex2 - m * m + 1e-5)
    x = (x1[...] - m) * (inv * g1[...]) + be1[...]
    x = jnp.maximum(jnp.dot(x, w2[...], preferred_element_type=jnp.float32) + b2[...], 0.0)
    y[...] = x
    ps2[...] = jnp.sum(x, axis=0, keepdims=True).reshape(1, 1, H)
    psq2[...] = jnp.sum(x * x, axis=0, keepdims=True).reshape(1, 1, H)


def _mlp3_body(y, ps2, psq2, g2, be2, w3, b3, w4r, b4, out):
    m = jnp.sum(ps2[...].reshape(NBLK, H), axis=0, keepdims=True) * (1.0 / B)
    ex2 = jnp.sum(psq2[...].reshape(NBLK, H), axis=0, keepdims=True) * (1.0 / B)
    inv = lax.rsqrt(ex2 - m * m + 1e-5)
    x = (y[...] - m) * (inv * g2[...]) + be2[...]
    x = jnp.maximum(jnp.dot(x, w3[...], preferred_element_type=jnp.float32) + b3[...], 0.0)
    o = jnp.sum(x * w4r[...], axis=1, keepdims=True) + b4[...]
    out[...] = 1.0 / (1.0 + jnp.exp(-o))


def _row(shape):
    return pl.BlockSpec(shape, lambda i: (0, 0))


def _blk(shape):
    return pl.BlockSpec(shape, lambda i: (i, 0))


_PSUM_OUT = pl.BlockSpec((1, 1, H), lambda i: (i, 0, 0))
_PSUM_IN = pl.BlockSpec((NBLK, 1, H), lambda i: (0, 0, 0))


def _mlp(se, ne, W1, b1, g1, be1, W2, b2, g2, be2, W3, b3, W4, b4):
    f32 = jnp.float32
    w1a = jnp.zeros((DP, H), f32).at[:D].set(W1[:D])
    w1b = W1[D:]
    b1r = b1.reshape(1, H)
    g1r = g1.reshape(1, H)
    be1r = be1.reshape(1, H)
    b2r = b2.reshape(1, H)
    g2r = g2.reshape(1, H)
    be2r = be2.reshape(1, H)
    b3r = b3.reshape(1, H)
    w4r = W4.reshape(1, H)
    b4r = b4.reshape(1, 1)

    x1, ps, psq = pl.pallas_call(
        _mlp1_body,
        grid=(NBLK,),
        in_specs=[_blk((BBLK, DP)), _blk((BBLK, D)), _row((DP, H)), _row((D, H)),
                  _row((1, H))],
        out_specs=[_blk((BBLK, H)), _PSUM_OUT, _PSUM_OUT],
        out_shape=[jax.ShapeDtypeStruct((B, H), f32),
                   jax.ShapeDtypeStruct((NBLK, 1, H), f32),
                   jax.ShapeDtypeStruct((NBLK, 1, H), f32)],
    )(se, ne, w1a, w1b, b1r)

    y, ps2, psq2 = pl.pallas_call(
        _mlp2_body,
        grid=(NBLK,),
        in_specs=[_blk((BBLK, H)), _PSUM_IN, _PSUM_IN,
                  _row((1, H)), _row((1, H)), _row((H, H)), _row((1, H))],
        out_specs=[_blk((BBLK, H)), _PSUM_OUT, _PSUM_OUT],
        out_shape=[jax.ShapeDtypeStruct((B, H), f32),
                   jax.ShapeDtypeStruct((NBLK, 1, H), f32),
                   jax.ShapeDtypeStruct((NBLK, 1, H), f32)],
    )(x1, ps, psq, g1r, be1r, W2, b2r)

    out = pl.pallas_call(
        _mlp3_body,
        grid=(NBLK,),
        in_specs=[_blk((BBLK, H)), _PSUM_IN, _PSUM_IN,
                  _row((1, H)), _row((1, H)), _row((H, H)), _row((1, H)),
                  _row((1, H)), _row((1, 1))],
        out_specs=_blk((BBLK, 1)),
        out_shape=jax.ShapeDtypeStruct((B, 1), f32),
    )(y, ps2, psq2, g2r, be2r, W3, b3r, w4r, b4r)

    return out.reshape(B)


def kernel(source, node, emb, W1, b1, g1, be1, W2, b2, g2, be2, W3, b3, W4, b4):
    emb_rm = _transpose_table(jnp.transpose(emb))
    se, ne = _sc_gather_mean(source, node, emb_rm)
    return se[:, 0] + ne[:, 0]
    se, ne = _sc_gather_mean(source, node, emb_rm)
    return _mlp(se, ne, W1, b1, g1, be1, W2, b2, g2, be2, W3, b3, W4, b4)


# speedup vs baseline: 1.0482x; 1.0482x over previous
import jax
import jax.numpy as jnp
from jax import lax
from jax.experimental import pallas as pl
from jax.experimental.pallas import tpu as pltpu
from jax.experimental.pallas import tpu_sc as plsc

B = 16384
HIST = 50
D = 200
DP = 208
H = 256
NW = 32
BPW = B // NW
OCH = 64
NCH = 128
LANES = 16
NFULL = D // LANES


RPG = 2
NSTEP = BPW // RPG
NBUF = 3


def _sc_body(src_hbm, node_hbm, emb_hbm, se_hbm, ne_hbm,
             idx_all, buf0, buf1, buf2, out_v, idxn, nbuf,
             sem0, sem1, sem2, semn):
    wid = lax.axis_index("s") * 2 + lax.axis_index("c")
    wbase = pl.multiple_of(wid * BPW, BPW)

    pltpu.sync_copy(src_hbm.at[pl.ds(wid * NSTEP, NSTEP), :], idx_all)

    pltpu.sync_copy(node_hbm.at[pl.ds(wbase, BPW)], idxn)

    def node_chunk(g, c):
        off = pl.multiple_of(g * NCH, NCH)
        pltpu.async_copy(emb_hbm.at[idxn.at[pl.ds(off, NCH)]], nbuf, semn).wait()
        pltpu.sync_copy(nbuf, ne_hbm.at[pl.ds(pl.multiple_of(wbase + off, NCH), NCH), :])
        return c

    lax.fori_loop(0, BPW // NCH, node_chunk, 0)

    zeros16 = jnp.zeros((LANES,), jnp.float32)

    def zinit(r, c):
        out_v[r, pl.ds(NFULL * LANES, LANES)] = zeros16
        return c

    lax.fori_loop(0, OCH, zinit, 0)

    bufs = (buf0, buf1, buf2)
    sems = (sem0, sem1, sem2)

    def start(t, k):
        pltpu.async_copy(emb_hbm.at[idx_all.at[t]], bufs[k], sems[k])

    def wait(t, k):
        pltpu.make_async_copy(emb_hbm.at[idx_all.at[t]], bufs[k], sems[k]).wait()

    def reduce_rows(t, k):
        buf = bufs[k]
        r = lax.rem(t * RPG, OCH)
        for p in range(RPG):
            def red(j, accs, p=p):
                new = [accs[c] + buf[p * HIST + j, pl.ds(c * LANES, LANES)]
                       for c in range(NFULL)]
                new.append(accs[NFULL] + buf[p * HIST + j, pl.ds(D - LANES, LANES)])
                return tuple(new)

            accs = lax.fori_loop(
                0, HIST, red,
                tuple(jnp.zeros((LANES,), jnp.float32) for _ in range(NFULL + 1)),
            )
            for c in range(NFULL):
                out_v[r + p, pl.ds(c * LANES, LANES)] = accs[c]
            out_v[r + p, pl.ds(D - LANES, LANES)] = accs[NFULL]

    for k in range(NBUF):
        start(k, k)

    def step_k(k):
        def go(t, c):
            wait(t, k)
            reduce_rows(t, k)

            @pl.when(t + NBUF < NSTEP)
            def _():
                start(t + NBUF, k)

            @pl.when(lax.rem(t * RPG, OCH) == OCH - RPG)
            def _():
                off = pl.multiple_of(wbase + t * RPG - (OCH - RPG), OCH)
                pltpu.sync_copy(out_v, se_hbm.at[pl.ds(off, OCH), :])

            return c
        return go

    def steps(g, c):
        t0 = g * NBUF
        for k in range(NBUF):
            c = step_k(k)(t0 + k, c)
        return c

    main = (NSTEP // NBUF) * NBUF
    lax.fori_loop(0, NSTEP // NBUF, steps, 0)
    for k in range(NSTEP - main):
        step_k(k)(main + k, 0)


def _sc_gather_mean(source, node, emb):
    mesh = plsc.VectorSubcoreMesh(core_axis_name="c", subcore_axis_name="s")
    f32 = jnp.float32
    run = pl.kernel(
        _sc_body,
        out_type=(
            jax.ShapeDtypeStruct((B, DP), f32),
            jax.ShapeDtypeStruct((B, D), f32),
        ),
        mesh=mesh,
        scratch_types=[
            pltpu.VMEM((NSTEP, RPG * HIST), jnp.int32),
            pltpu.VMEM((RPG * HIST, D), f32),
            pltpu.VMEM((RPG * HIST, D), f32),
            pltpu.VMEM((RPG * HIST, D), f32),
            pltpu.VMEM((OCH, DP), f32),
            pltpu.VMEM((BPW,), jnp.int32),
            pltpu.VMEM((NCH, D), f32),
            pltpu.SemaphoreType.DMA,
            pltpu.SemaphoreType.DMA,
            pltpu.SemaphoreType.DMA,
            pltpu.SemaphoreType.DMA,
        ],
        compiler_params=pltpu.CompilerParams(use_tc_tiling_on_sc=False),
    )
    return run(source.reshape(B // RPG, RPG * HIST), node, emb)


NN = 1000000
TBLK = 2048


def _transpose_body(src, dst):
    dst[...] = src[...].T


def _transpose_table(embT):
    return pl.pallas_call(
        _transpose_body,
        grid=(pl.cdiv(NN, TBLK),),
        in_specs=[pl.BlockSpec((D, TBLK), lambda i: (0, i))],
        out_specs=pl.BlockSpec((TBLK, D), lambda i: (i, 0)),
        out_shape=jax.ShapeDtypeStruct((NN, D), jnp.float32),
    )(embT)


BBLK = 2048
NBLK = B // BBLK


def _mlp1_body(se, ne, w1a, w1b, b1, x1, ps, psq):
    x = jnp.dot(se[...], w1a[...], preferred_element_type=jnp.float32)
    x = x + jnp.dot(ne[...], w1b[...], preferred_element_type=jnp.float32)
    x = jnp.maximum(x + b1[...], 0.0)
    x1[...] = x
    ps[...] = jnp.sum(x, axis=0, keepdims=True).reshape(1, 1, H)
    psq[...] = jnp.sum(x * x, axis=0, keepdims=True).reshape(1, 1, H)


def _mlp2_body(x1, ps, psq, g1, be1, w2, b2, y, ps2, psq2):
    m = jnp.sum(ps[...].reshape(NBLK, H), axis=0, keepdims=True) * (1.0 / B)
    ex2 = jnp.sum(psq[...].reshape(NBLK, H), axis=0, keepdims=True) * (1.0 / B)
    inv = lax.rsqrt(ex2 - m * m + 1e-5)
    x = (x1[...] - m) * (inv * g1[...]) + be1[...]
    x = jnp.maximum(jnp.dot(x, w2[...], preferred_element_type=jnp.float32) + b2[...], 0.0)
    y[...] = x
    ps2[...] = jnp.sum(x, axis=0, keepdims=True).reshape(1, 1, H)
    psq2[...] = jnp.sum(x * x, axis=0, keepdims=True).reshape(1, 1, H)


def _mlp3_body(y, ps2, psq2, g2, be2, w3, b3, w4r, b4, out):
    m = jnp.sum(ps2[...].reshape(NBLK, H), axis=0, keepdims=True) * (1.0 / B)
    ex2 = jnp.sum(psq2[...].reshape(NBLK, H), axis=0, keepdims=True) * (1.0 / B)
    inv = lax.rsqrt(ex2 - m * m + 1e-5)
    x = (y[...] - m) * (inv * g2[...]) + be2[...]
    x = jnp.maximum(jnp.dot(x, w3[...], preferred_element_type=jnp.float32) + b3[...], 0.0)
    o = jnp.sum(x * w4r[...], axis=1, keepdims=True) + b4[...]
    out[...] = 1.0 / (1.0 + jnp.exp(-o))


def _row(shape):
    return pl.BlockSpec(shape, lambda i: (0, 0))


def _blk(shape):
    return pl.BlockSpec(shape, lambda i: (i, 0))


_PSUM_OUT = pl.BlockSpec((1, 1, H), lambda i: (i, 0, 0))
_PSUM_IN = pl.BlockSpec((NBLK, 1, H), lambda i: (0, 0, 0))


def _mlp(se, ne, W1, b1, g1, be1, W2, b2, g2, be2, W3, b3, W4, b4):
    f32 = jnp.float32
    w1a = jnp.zeros((DP, H), f32).at[:D].set(W1[:D] * (1.0 / HIST))
    w1b = W1[D:]
    b1r = b1.reshape(1, H)
    g1r = g1.reshape(1, H)
    be1r = be1.reshape(1, H)
    b2r = b2.reshape(1, H)
    g2r = g2.reshape(1, H)
    be2r = be2.reshape(1, H)
    b3r = b3.reshape(1, H)
    w4r = W4.reshape(1, H)
    b4r = b4.reshape(1, 1)

    x1, ps, psq = pl.pallas_call(
        _mlp1_body,
        grid=(NBLK,),
        in_specs=[_blk((BBLK, DP)), _blk((BBLK, D)), _row((DP, H)), _row((D, H)),
                  _row((1, H))],
        out_specs=[_blk((BBLK, H)), _PSUM_OUT, _PSUM_OUT],
        out_shape=[jax.ShapeDtypeStruct((B, H), f32),
                   jax.ShapeDtypeStruct((NBLK, 1, H), f32),
                   jax.ShapeDtypeStruct((NBLK, 1, H), f32)],
    )(se, ne, w1a, w1b, b1r)

    y, ps2, psq2 = pl.pallas_call(
        _mlp2_body,
        grid=(NBLK,),
        in_specs=[_blk((BBLK, H)), _PSUM_IN, _PSUM_IN,
                  _row((1, H)), _row((1, H)), _row((H, H)), _row((1, H))],
        out_specs=[_blk((BBLK, H)), _PSUM_OUT, _PSUM_OUT],
        out_shape=[jax.ShapeDtypeStruct((B, H), f32),
                   jax.ShapeDtypeStruct((NBLK, 1, H), f32),
                   jax.ShapeDtypeStruct((NBLK, 1, H), f32)],
    )(x1, ps, psq, g1r, be1r, W2, b2r)

    out = pl.pallas_call(
        _mlp3_body,
        grid=(NBLK,),
        in_specs=[_blk((BBLK, H)), _PSUM_IN, _PSUM_IN,
                  _row((1, H)), _row((1, H)), _row((H, H)), _row((1, H)),
                  _row((1, H)), _row((1, 1))],
        out_specs=_blk((BBLK, 1)),
        out_shape=jax.ShapeDtypeStruct((B, 1), f32),
    )(y, ps2, psq2, g2r, be2r, W3, b3r, w4r, b4r)

    return out.reshape(B)


def kernel(source, node, emb, W1, b1, g1, be1, W2, b2, g2, be2, W3, b3, W4, b4):
    emb_rm = _transpose_table(jnp.transpose(emb))
    se, ne = _sc_gather_mean(source, node, emb_rm)
    se, ne = _sc_gather_mean(source, node, emb_rm)
    return _mlp(se, ne, W1, b1, g1, be1, W2, b2, g2, be2, W3, b3, W4, b4)


# speedup vs baseline: 1.1433x; 1.0907x over previous
import jax
import jax.numpy as jnp
from jax import lax
from jax.experimental import pallas as pl
from jax.experimental.pallas import tpu as pltpu
from jax.experimental.pallas import tpu_sc as plsc

B = 16384
HIST = 50
D = 200
DP = 208
H = 256
NW = 32
BPW = B // NW
OCH = 64
NCH = 128
LANES = 16
NFULL = D // LANES


RPG = 2
NSTEP = BPW // RPG
NBUF = 3


def _sc_body(src_hbm, node_hbm, emb_hbm, se_hbm, ne_hbm,
             idx_all, buf0, buf1, buf2, out_v, idxn, nbuf,
             sem0, sem1, sem2, semn):
    wid = lax.axis_index("s") * 2 + lax.axis_index("c")
    wbase = pl.multiple_of(wid * BPW, BPW)

    pltpu.sync_copy(src_hbm.at[pl.ds(wid * NSTEP, NSTEP), :], idx_all)

    pltpu.sync_copy(node_hbm.at[pl.ds(wbase, BPW)], idxn)

    def node_chunk(g, c):
        off = pl.multiple_of(g * NCH, NCH)
        pltpu.async_copy(emb_hbm.at[idxn.at[pl.ds(off, NCH)]], nbuf, semn).wait()
        pltpu.sync_copy(nbuf, ne_hbm.at[pl.ds(pl.multiple_of(wbase + off, NCH), NCH), :])
        return c

    lax.fori_loop(0, BPW // NCH, node_chunk, 0)

    zeros16 = jnp.zeros((LANES,), jnp.float32)

    def zinit(r, c):
        out_v[r, pl.ds(NFULL * LANES, LANES)] = zeros16
        return c

    lax.fori_loop(0, OCH, zinit, 0)

    bufs = (buf0, buf1, buf2)
    sems = (sem0, sem1, sem2)

    def start(t, k):
        pltpu.async_copy(emb_hbm.at[idx_all.at[t]], bufs[k], sems[k])

    def wait(t, k):
        pltpu.make_async_copy(emb_hbm.at[idx_all.at[t]], bufs[k], sems[k]).wait()

    def reduce_rows(t, k):
        buf = bufs[k]
        r = lax.rem(t * RPG, OCH)
        for p in range(RPG):
            def red(j, accs, p=p):
                new = [accs[c] + buf[p * HIST + j, pl.ds(c * LANES, LANES)]
                       for c in range(NFULL)]
                new.append(accs[NFULL] + buf[p * HIST + j, pl.ds(D - LANES, LANES)])
                return tuple(new)

            accs = lax.fori_loop(
                0, HIST, red,
                tuple(jnp.zeros((LANES,), jnp.float32) for _ in range(NFULL + 1)),
            )
            for c in range(NFULL):
                out_v[r + p, pl.ds(c * LANES, LANES)] = accs[c]
            out_v[r + p, pl.ds(D - LANES, LANES)] = accs[NFULL]

    for k in range(NBUF):
        start(k, k)

    def step_k(k):
        def go(t, c):
            wait(t, k)
            reduce_rows(t, k)

            @pl.when(t + NBUF < 24)
            def _():
                start(t + NBUF, k)

            @pl.when(lax.rem(t * RPG, OCH) == OCH - RPG)
            def _():
                off = pl.multiple_of(wbase + t * RPG - (OCH - RPG), OCH)
                pltpu.sync_copy(out_v, se_hbm.at[pl.ds(off, OCH), :])

            return c
        return go

    def steps(g, c):
        t0 = g * NBUF
        for k in range(NBUF):
            c = step_k(k)(t0 + k, c)
        return c

    main = (NSTEP // NBUF) * NBUF
    lax.fori_loop(0, 8, steps, 0)


def _sc_gather_mean(source, node, emb):
    mesh = plsc.VectorSubcoreMesh(core_axis_name="c", subcore_axis_name="s")
    f32 = jnp.float32
    run = pl.kernel(
        _sc_body,
        out_type=(
            jax.ShapeDtypeStruct((B, DP), f32),
            jax.ShapeDtypeStruct((B, D), f32),
        ),
        mesh=mesh,
        scratch_types=[
            pltpu.VMEM((NSTEP, RPG * HIST), jnp.int32),
            pltpu.VMEM((RPG * HIST, D), f32),
            pltpu.VMEM((RPG * HIST, D), f32),
            pltpu.VMEM((RPG * HIST, D), f32),
            pltpu.VMEM((OCH, DP), f32),
            pltpu.VMEM((BPW,), jnp.int32),
            pltpu.VMEM((NCH, D), f32),
            pltpu.SemaphoreType.DMA,
            pltpu.SemaphoreType.DMA,
            pltpu.SemaphoreType.DMA,
            pltpu.SemaphoreType.DMA,
        ],
        compiler_params=pltpu.CompilerParams(use_tc_tiling_on_sc=False),
    )
    return run(source.reshape(B // RPG, RPG * HIST), node, emb)


NN = 1000000
TBLK = 2048


def _transpose_body(src, dst):
    dst[...] = src[...].T


def _transpose_table(embT):
    return pl.pallas_call(
        _transpose_body,
        grid=(pl.cdiv(NN, TBLK),),
        in_specs=[pl.BlockSpec((D, TBLK), lambda i: (0, i))],
        out_specs=pl.BlockSpec((TBLK, D), lambda i: (i, 0)),
        out_shape=jax.ShapeDtypeStruct((NN, D), jnp.float32),
    )(embT)


BBLK = 2048
NBLK = B // BBLK


def _mlp1_body(se, ne, w1a, w1b, b1, x1, ps, psq):
    x = jnp.dot(se[...], w1a[...], preferred_element_type=jnp.float32)
    x = x + jnp.dot(ne[...], w1b[...], preferred_element_type=jnp.float32)
    x = jnp.maximum(x + b1[...], 0.0)
    x1[...] = x
    ps[...] = jnp.sum(x, axis=0, keepdims=True).reshape(1, 1, H)
    psq[...] = jnp.sum(x * x, axis=0, keepdims=True).reshape(1, 1, H)


def _mlp2_body(x1, ps, psq, g1, be1, w2, b2, y, ps2, psq2):
    m = jnp.sum(ps[...].reshape(NBLK, H), axis=0, keepdims=True) * (1.0 / B)
    ex2 = jnp.sum(psq[...].reshape(NBLK, H), axis=0, keepdims=True) * (1.0 / B)
    inv = lax.rsqrt(ex2 - m * m + 1e-5)
    x = (x1[...] - m) * (inv * g1[...]) + be1[...]
    x = jnp.maximum(jnp.dot(x, w2[...], preferred_element_type=jnp.float32) + b2[...], 0.0)
    y[...] = x
    ps2[...] = jnp.sum(x, axis=0, keepdims=True).reshape(1, 1, H)
    psq2[...] = jnp.sum(x * x, axis=0, keepdims=True).reshape(1, 1, H)


def _mlp3_body(y, ps2, psq2, g2, be2, w3, b3, w4r, b4, out):
    m = jnp.sum(ps2[...].reshape(NBLK, H), axis=0, keepdims=True) * (1.0 / B)
    ex2 = jnp.sum(psq2[...].reshape(NBLK, H), axis=0, keepdims=True) * (1.0 / B)
    inv = lax.rsqrt(ex2 - m * m + 1e-5)
    x = (y[...] - m) * (inv * g2[...]) + be2[...]
    x = jnp.maximum(jnp.dot(x, w3[...], preferred_element_type=jnp.float32) + b3[...], 0.0)
    o = jnp.sum(x * w4r[...], axis=1, keepdims=True) + b4[...]
    out[...] = 1.0 / (1.0 + jnp.exp(-o))


def _row(shape):
    return pl.BlockSpec(shape, lambda i: (0, 0))


def _blk(shape):
    return pl.BlockSpec(shape, lambda i: (i, 0))


_PSUM_OUT = pl.BlockSpec((1, 1, H), lambda i: (i, 0, 0))
_PSUM_IN = pl.BlockSpec((NBLK, 1, H), lambda i: (0, 0, 0))


def _mlp(se, ne, W1, b1, g1, be1, W2, b2, g2, be2, W3, b3, W4, b4):
    f32 = jnp.float32
    w1a = jnp.zeros((DP, H), f32).at[:D].set(W1[:D] * (1.0 / HIST))
    w1b = W1[D:]
    b1r = b1.reshape(1, H)
    g1r = g1.reshape(1, H)
    be1r = be1.reshape(1, H)
    b2r = b2.reshape(1, H)
    g2r = g2.reshape(1, H)
    be2r = be2.reshape(1, H)
    b3r = b3.reshape(1, H)
    w4r = W4.reshape(1, H)
    b4r = b4.reshape(1, 1)

    x1, ps, psq = pl.pallas_call(
        _mlp1_body,
        grid=(NBLK,),
        in_specs=[_blk((BBLK, DP)), _blk((BBLK, D)), _row((DP, H)), _row((D, H)),
                  _row((1, H))],
        out_specs=[_blk((BBLK, H)), _PSUM_OUT, _PSUM_OUT],
        out_shape=[jax.ShapeDtypeStruct((B, H), f32),
                   jax.ShapeDtypeStruct((NBLK, 1, H), f32),
                   jax.ShapeDtypeStruct((NBLK, 1, H), f32)],
    )(se, ne, w1a, w1b, b1r)

    y, ps2, psq2 = pl.pallas_call(
        _mlp2_body,
        grid=(NBLK,),
        in_specs=[_blk((BBLK, H)), _PSUM_IN, _PSUM_IN,
                  _row((1, H)), _row((1, H)), _row((H, H)), _row((1, H))],
        out_specs=[_blk((BBLK, H)), _PSUM_OUT, _PSUM_OUT],
        out_shape=[jax.ShapeDtypeStruct((B, H), f32),
                   jax.ShapeDtypeStruct((NBLK, 1, H), f32),
                   jax.ShapeDtypeStruct((NBLK, 1, H), f32)],
    )(x1, ps, psq, g1r, be1r, W2, b2r)

    out = pl.pallas_call(
        _mlp3_body,
        grid=(NBLK,),
        in_specs=[_blk((BBLK, H)), _PSUM_IN, _PSUM_IN,
                  _row((1, H)), _row((1, H)), _row((H, H)), _row((1, H)),
                  _row((1, H)), _row((1, 1))],
        out_specs=_blk((BBLK, 1)),
        out_shape=jax.ShapeDtypeStruct((B, 1), f32),
    )(y, ps2, psq2, g2r, be2r, W3, b3r, w4r, b4r)

    return out.reshape(B)


def kernel(source, node, emb, W1, b1, g1, be1, W2, b2, g2, be2, W3, b3, W4, b4):
    emb_rm = _transpose_table(jnp.transpose(emb))
    se, ne = _sc_gather_mean(source, node, emb_rm)
    se, ne = _sc_gather_mean(source, node, emb_rm)
    return _mlp(se, ne, W1, b1, g1, be1, W2, b2, g2, be2, W3, b3, W4, b4)


# speedup vs baseline: 1.1487x; 1.0048x over previous
import jax
import jax.numpy as jnp
from jax import lax
from jax.experimental import pallas as pl
from jax.experimental.pallas import tpu as pltpu
from jax.experimental.pallas import tpu_sc as plsc

B = 16384
HIST = 50
D = 200
DP = 208
H = 256
NW = 32
BPW = B // NW
OCH = 64
NCH = 128
LANES = 16
NFULL = D // LANES


RPG = 2
NSTEP = BPW // RPG
NBUF = 3


def _sc_body(src_hbm, node_hbm, emb_hbm, se_hbm, ne_hbm,
             idx_all, buf0, buf1, buf2, out_v, idxn, nbuf,
             sem0, sem1, sem2, semn):
    wid = lax.axis_index("s") * 2 + lax.axis_index("c")
    wbase = pl.multiple_of(wid * BPW, BPW)

    pltpu.sync_copy(src_hbm.at[pl.ds(wid * NSTEP, NSTEP), :], idx_all)

    pltpu.sync_copy(node_hbm.at[pl.ds(wbase, BPW)], idxn)

    def node_chunk(g, c):
        off = pl.multiple_of(g * NCH, NCH)
        pltpu.async_copy(emb_hbm.at[idxn.at[pl.ds(off, NCH)]], nbuf, semn).wait()
        pltpu.sync_copy(nbuf, ne_hbm.at[pl.ds(pl.multiple_of(wbase + off, NCH), NCH), :])
        return c

    lax.fori_loop(0, 1, node_chunk, 0)

    zeros16 = jnp.zeros((LANES,), jnp.float32)

    def zinit(r, c):
        out_v[r, pl.ds(NFULL * LANES, LANES)] = zeros16
        return c

    lax.fori_loop(0, OCH, zinit, 0)

    bufs = (buf0, buf1, buf2)
    sems = (sem0, sem1, sem2)

    def start(t, k):
        pltpu.async_copy(emb_hbm.at[idx_all.at[t]], bufs[k], sems[k])

    def wait(t, k):
        pltpu.make_async_copy(emb_hbm.at[idx_all.at[t]], bufs[k], sems[k]).wait()

    def reduce_rows(t, k):
        buf = bufs[k]
        r = lax.rem(t * RPG, OCH)
        for p in range(RPG):
            def red(j, accs, p=p):
                new = [accs[c] + buf[p * HIST + j, pl.ds(c * LANES, LANES)]
                       for c in range(NFULL)]
                new.append(accs[NFULL] + buf[p * HIST + j, pl.ds(D - LANES, LANES)])
                return tuple(new)

            accs = lax.fori_loop(
                0, HIST, red,
                tuple(jnp.zeros((LANES,), jnp.float32) for _ in range(NFULL + 1)),
            )
            for c in range(NFULL):
                out_v[r + p, pl.ds(c * LANES, LANES)] = accs[c]
            out_v[r + p, pl.ds(D - LANES, LANES)] = accs[NFULL]

    for k in range(NBUF):
        start(k, k)

    def step_k(k):
        def go(t, c):
            wait(t, k)
            reduce_rows(t, k)

            @pl.when(t + NBUF < 24)
            def _():
                start(t + NBUF, k)

            @pl.when(lax.rem(t * RPG, OCH) == OCH - RPG)
            def _():
                off = pl.multiple_of(wbase + t * RPG - (OCH - RPG), OCH)
                pltpu.sync_copy(out_v, se_hbm.at[pl.ds(off, OCH), :])

            return c
        return go

    def steps(g, c):
        t0 = g * NBUF
        for k in range(NBUF):
            c = step_k(k)(t0 + k, c)
        return c

    main = (NSTEP // NBUF) * NBUF
    lax.fori_loop(0, 8, steps, 0)


def _sc_gather_mean(source, node, emb):
    mesh = plsc.VectorSubcoreMesh(core_axis_name="c", subcore_axis_name="s")
    f32 = jnp.float32
    run = pl.kernel(
        _sc_body,
        out_type=(
            jax.ShapeDtypeStruct((B, DP), f32),
            jax.ShapeDtypeStruct((B, D), f32),
        ),
        mesh=mesh,
        scratch_types=[
            pltpu.VMEM((NSTEP, RPG * HIST), jnp.int32),
            pltpu.VMEM((RPG * HIST, D), f32),
            pltpu.VMEM((RPG * HIST, D), f32),
            pltpu.VMEM((RPG * HIST, D), f32),
            pltpu.VMEM((OCH, DP), f32),
            pltpu.VMEM((BPW,), jnp.int32),
            pltpu.VMEM((NCH, D), f32),
            pltpu.SemaphoreType.DMA,
            pltpu.SemaphoreType.DMA,
            pltpu.SemaphoreType.DMA,
            pltpu.SemaphoreType.DMA,
        ],
        compiler_params=pltpu.CompilerParams(use_tc_tiling_on_sc=False),
    )
    return run(source.reshape(B // RPG, RPG * HIST), node, emb)


NN = 1000000
TBLK = 2048


def _transpose_body(src, dst):
    dst[...] = src[...].T


def _transpose_table(embT):
    return pl.pallas_call(
        _transpose_body,
        grid=(pl.cdiv(NN, TBLK),),
        in_specs=[pl.BlockSpec((D, TBLK), lambda i: (0, i))],
        out_specs=pl.BlockSpec((TBLK, D), lambda i: (i, 0)),
        out_shape=jax.ShapeDtypeStruct((NN, D), jnp.float32),
    )(embT)


BBLK = 2048
NBLK = B // BBLK


def _mlp1_body(se, ne, w1a, w1b, b1, x1, ps, psq):
    x = jnp.dot(se[...], w1a[...], preferred_element_type=jnp.float32)
    x = x + jnp.dot(ne[...], w1b[...], preferred_element_type=jnp.float32)
    x = jnp.maximum(x + b1[...], 0.0)
    x1[...] = x
    ps[...] = jnp.sum(x, axis=0, keepdims=True).reshape(1, 1, H)
    psq[...] = jnp.sum(x * x, axis=0, keepdims=True).reshape(1, 1, H)


def _mlp2_body(x1, ps, psq, g1, be1, w2, b2, y, ps2, psq2):
    m = jnp.sum(ps[...].reshape(NBLK, H), axis=0, keepdims=True) * (1.0 / B)
    ex2 = jnp.sum(psq[...].reshape(NBLK, H), axis=0, keepdims=True) * (1.0 / B)
    inv = lax.rsqrt(ex2 - m * m + 1e-5)
    x = (x1[...] - m) * (inv * g1[...]) + be1[...]
    x = jnp.maximum(jnp.dot(x, w2[...], preferred_element_type=jnp.float32) + b2[...], 0.0)
    y[...] = x
    ps2[...] = jnp.sum(x, axis=0, keepdims=True).reshape(1, 1, H)
    psq2[...] = jnp.sum(x * x, axis=0, keepdims=True).reshape(1, 1, H)


def _mlp3_body(y, ps2, psq2, g2, be2, w3, b3, w4r, b4, out):
    m = jnp.sum(ps2[...].reshape(NBLK, H), axis=0, keepdims=True) * (1.0 / B)
    ex2 = jnp.sum(psq2[...].reshape(NBLK, H), axis=0, keepdims=True) * (1.0 / B)
    inv = lax.rsqrt(ex2 - m * m + 1e-5)
    x = (y[...] - m) * (inv * g2[...]) + be2[...]
    x = jnp.maximum(jnp.dot(x, w3[...], preferred_element_type=jnp.float32) + b3[...], 0.0)
    o = jnp.sum(x * w4r[...], axis=1, keepdims=True) + b4[...]
    out[...] = 1.0 / (1.0 + jnp.exp(-o))


def _row(shape):
    return pl.BlockSpec(shape, lambda i: (0, 0))


def _blk(shape):
    return pl.BlockSpec(shape, lambda i: (i, 0))


_PSUM_OUT = pl.BlockSpec((1, 1, H), lambda i: (i, 0, 0))
_PSUM_IN = pl.BlockSpec((NBLK, 1, H), lambda i: (0, 0, 0))


def _mlp(se, ne, W1, b1, g1, be1, W2, b2, g2, be2, W3, b3, W4, b4):
    f32 = jnp.float32
    w1a = jnp.zeros((DP, H), f32).at[:D].set(W1[:D] * (1.0 / HIST))
    w1b = W1[D:]
    b1r = b1.reshape(1, H)
    g1r = g1.reshape(1, H)
    be1r = be1.reshape(1, H)
    b2r = b2.reshape(1, H)
    g2r = g2.reshape(1, H)
    be2r = be2.reshape(1, H)
    b3r = b3.reshape(1, H)
    w4r = W4.reshape(1, H)
    b4r = b4.reshape(1, 1)

    x1, ps, psq = pl.pallas_call(
        _mlp1_body,
        grid=(NBLK,),
        in_specs=[_blk((BBLK, DP)), _blk((BBLK, D)), _row((DP, H)), _row((D, H)),
                  _row((1, H))],
        out_specs=[_blk((BBLK, H)), _PSUM_OUT, _PSUM_OUT],
        out_shape=[jax.ShapeDtypeStruct((B, H), f32),
                   jax.ShapeDtypeStruct((NBLK, 1, H), f32),
                   jax.ShapeDtypeStruct((NBLK, 1, H), f32)],
    )(se, ne, w1a, w1b, b1r)

    y, ps2, psq2 = pl.pallas_call(
        _mlp2_body,
        grid=(NBLK,),
        in_specs=[_blk((BBLK, H)), _PSUM_IN, _PSUM_IN,
                  _row((1, H)), _row((1, H)), _row((H, H)), _row((1, H))],
        out_specs=[_blk((BBLK, H)), _PSUM_OUT, _PSUM_OUT],
        out_shape=[jax.ShapeDtypeStruct((B, H), f32),
                   jax.ShapeDtypeStruct((NBLK, 1, H), f32),
                   jax.ShapeDtypeStruct((NBLK, 1, H), f32)],
    )(x1, ps, psq, g1r, be1r, W2, b2r)

    out = pl.pallas_call(
        _mlp3_body,
        grid=(NBLK,),
        in_specs=[_blk((BBLK, H)), _PSUM_IN, _PSUM_IN,
                  _row((1, H)), _row((1, H)), _row((H, H)), _row((1, H)),
                  _row((1, H)), _row((1, 1))],
        out_specs=_blk((BBLK, 1)),
        out_shape=jax.ShapeDtypeStruct((B, 1), f32),
    )(y, ps2, psq2, g2r, be2r, W3, b3r, w4r, b4r)

    return out.reshape(B)


def kernel(source, node, emb, W1, b1, g1, be1, W2, b2, g2, be2, W3, b3, W4, b4):
    emb_rm = _transpose_table(jnp.transpose(emb))
    se, ne = _sc_gather_mean(source, node, emb_rm)
    se, ne = _sc_gather_mean(source, node, emb_rm)
    return _mlp(se, ne, W1, b1, g1, be1, W2, b2, g2, be2, W3, b3, W4, b4)


# speedup vs baseline: 1.1494x; 1.0006x over previous
import jax
import jax.numpy as jnp
from jax import lax
from jax.experimental import pallas as pl
from jax.experimental.pallas import tpu as pltpu
from jax.experimental.pallas import tpu_sc as plsc

B = 16384
HIST = 50
D = 200
DP = 208
H = 256
NW = 32
BPW = B // NW
OCH = 64
NCH = 128
LANES = 16
NFULL = D // LANES


RPG = 2
NSTEP = BPW // RPG
NBUF = 3


def _sc_body(src_hbm, node_hbm, emb_hbm, se_hbm, ne_hbm,
             idx_all, buf0, buf1, buf2, out_v, idxn, nbuf,
             sem0, sem1, sem2, semn):
    wid = lax.axis_index("s") * 2 + lax.axis_index("c")
    wbase = pl.multiple_of(wid * BPW, BPW)

    pltpu.sync_copy(src_hbm.at[pl.ds(wid * NSTEP, NSTEP), :], idx_all)

    pltpu.sync_copy(node_hbm.at[pl.ds(wbase, BPW)], idxn)

    def node_chunk(g, c):
        off = pl.multiple_of(g * NCH, NCH)
        pltpu.async_copy(emb_hbm.at[idxn.at[pl.ds(off, NCH)]], nbuf, semn).wait()
        pltpu.sync_copy(nbuf, ne_hbm.at[pl.ds(pl.multiple_of(wbase + off, NCH), NCH), :])
        return c

    lax.fori_loop(0, 1, node_chunk, 0)

    zeros16 = jnp.zeros((LANES,), jnp.float32)

    def zinit(r, c):
        out_v[r, pl.ds(NFULL * LANES, LANES)] = zeros16
        return c

    lax.fori_loop(0, OCH, zinit, 0)

    bufs = (buf0, buf1, buf2)
    sems = (sem0, sem1, sem2)

    def start(t, k):
        pltpu.async_copy(emb_hbm.at[idx_all.at[t]], bufs[k], sems[k])

    def wait(t, k):
        pltpu.make_async_copy(emb_hbm.at[idx_all.at[t]], bufs[k], sems[k]).wait()

    def reduce_rows(t, k):
        buf = bufs[k]
        r = lax.rem(t * RPG, OCH)
        for p in range(RPG):
            def red(j, accs, p=p):
                new = [accs[c] + buf[p * HIST + j, pl.ds(c * LANES, LANES)]
                       for c in range(NFULL)]
                new.append(accs[NFULL] + buf[p * HIST + j, pl.ds(D - LANES, LANES)])
                return tuple(new)

            accs = lax.fori_loop(
                0, HIST, red,
                tuple(jnp.zeros((LANES,), jnp.float32) for _ in range(NFULL + 1)),
            )
            for c in range(NFULL):
                out_v[r + p, pl.ds(c * LANES, LANES)] = accs[c]
            out_v[r + p, pl.ds(D - LANES, LANES)] = accs[NFULL]

    for k in range(NBUF):
        start(k, k)

    def step_k(k):
        def go(t, c):
            wait(t, k)
            reduce_rows(t, k)

            @pl.when(t + NBUF < 24)
            def _():
                start(t + NBUF, k)

            @pl.when(lax.rem(t * RPG, OCH) == OCH - RPG)
            def _():
                off = pl.multiple_of(wbase + t * RPG - (OCH - RPG), OCH)
                pltpu.sync_copy(out_v, se_hbm.at[pl.ds(off, OCH), :])

            return c
        return go

    def steps(g, c):
        t0 = g * NBUF
        for k in range(NBUF):
            c = step_k(k)(t0 + k, c)
        return c

    main = (NSTEP // NBUF) * NBUF
    lax.fori_loop(0, 8, steps, 0)


def _sc_gather_mean(source, node, emb):
    mesh = plsc.VectorSubcoreMesh(core_axis_name="c", subcore_axis_name="s")
    f32 = jnp.float32
    run = pl.kernel(
        _sc_body,
        out_type=(
            jax.ShapeDtypeStruct((B, DP), f32),
            jax.ShapeDtypeStruct((B, D), f32),
        ),
        mesh=mesh,
        scratch_types=[
            pltpu.VMEM((NSTEP, RPG * HIST), jnp.int32),
            pltpu.VMEM((RPG * HIST, D), f32),
            pltpu.VMEM((RPG * HIST, D), f32),
            pltpu.VMEM((RPG * HIST, D), f32),
            pltpu.VMEM((OCH, DP), f32),
            pltpu.VMEM((BPW,), jnp.int32),
            pltpu.VMEM((NCH, D), f32),
            pltpu.SemaphoreType.DMA,
            pltpu.SemaphoreType.DMA,
            pltpu.SemaphoreType.DMA,
            pltpu.SemaphoreType.DMA,
        ],
        compiler_params=pltpu.CompilerParams(
            use_tc_tiling_on_sc=False, skip_device_barrier=True),
    )
    return run(source.reshape(B // RPG, RPG * HIST), node, emb)


NN = 1000000
TBLK = 2048


def _transpose_body(src, dst):
    dst[...] = src[...].T


def _transpose_table(embT):
    return pl.pallas_call(
        _transpose_body,
        grid=(pl.cdiv(NN, TBLK),),
        in_specs=[pl.BlockSpec((D, TBLK), lambda i: (0, i))],
        out_specs=pl.BlockSpec((TBLK, D), lambda i: (i, 0)),
        out_shape=jax.ShapeDtypeStruct((NN, D), jnp.float32),
    )(embT)


BBLK = 2048
NBLK = B // BBLK


def _mlp1_body(se, ne, w1a, w1b, b1, x1, ps, psq):
    x = jnp.dot(se[...], w1a[...], preferred_element_type=jnp.float32)
    x = x + jnp.dot(ne[...], w1b[...], preferred_element_type=jnp.float32)
    x = jnp.maximum(x + b1[...], 0.0)
    x1[...] = x
    ps[...] = jnp.sum(x, axis=0, keepdims=True).reshape(1, 1, H)
    psq[...] = jnp.sum(x * x, axis=0, keepdims=True).reshape(1, 1, H)


def _mlp2_body(x1, ps, psq, g1, be1, w2, b2, y, ps2, psq2):
    m = jnp.sum(ps[...].reshape(NBLK, H), axis=0, keepdims=True) * (1.0 / B)
    ex2 = jnp.sum(psq[...].reshape(NBLK, H), axis=0, keepdims=True) * (1.0 / B)
    inv = lax.rsqrt(ex2 - m * m + 1e-5)
    x = (x1[...] - m) * (inv * g1[...]) + be1[...]
    x = jnp.maximum(jnp.dot(x, w2[...], preferred_element_type=jnp.float32) + b2[...], 0.0)
    y[...] = x
    ps2[...] = jnp.sum(x, axis=0, keepdims=True).reshape(1, 1, H)
    psq2[...] = jnp.sum(x * x, axis=0, keepdims=True).reshape(1, 1, H)


def _mlp3_body(y, ps2, psq2, g2, be2, w3, b3, w4r, b4, out):
    m = jnp.sum(ps2[...].reshape(NBLK, H), axis=0, keepdims=True) * (1.0 / B)
    ex2 = jnp.sum(psq2[...].reshape(NBLK, H), axis=0, keepdims=True) * (1.0 / B)
    inv = lax.rsqrt(ex2 - m * m + 1e-5)
    x = (y[...] - m) * (inv * g2[...]) + be2[...]
    x = jnp.maximum(jnp.dot(x, w3[...], preferred_element_type=jnp.float32) + b3[...], 0.0)
    o = jnp.sum(x * w4r[...], axis=1, keepdims=True) + b4[...]
    out[...] = 1.0 / (1.0 + jnp.exp(-o))


def _row(shape):
    return pl.BlockSpec(shape, lambda i: (0, 0))


def _blk(shape):
    return pl.BlockSpec(shape, lambda i: (i, 0))


_PSUM_OUT = pl.BlockSpec((1, 1, H), lambda i: (i, 0, 0))
_PSUM_IN = pl.BlockSpec((NBLK, 1, H), lambda i: (0, 0, 0))


def _mlp(se, ne, W1, b1, g1, be1, W2, b2, g2, be2, W3, b3, W4, b4):
    f32 = jnp.float32
    w1a = jnp.zeros((DP, H), f32).at[:D].set(W1[:D] * (1.0 / HIST))
    w1b = W1[D:]
    b1r = b1.reshape(1, H)
    g1r = g1.reshape(1, H)
    be1r = be1.reshape(1, H)
    b2r = b2.reshape(1, H)
    g2r = g2.reshape(1, H)
    be2r = be2.reshape(1, H)
    b3r = b3.reshape(1, H)
    w4r = W4.reshape(1, H)
    b4r = b4.reshape(1, 1)

    x1, ps, psq = pl.pallas_call(
        _mlp1_body,
        grid=(NBLK,),
        in_specs=[_blk((BBLK, DP)), _blk((BBLK, D)), _row((DP, H)), _row((D, H)),
                  _row((1, H))],
        out_specs=[_blk((BBLK, H)), _PSUM_OUT, _PSUM_OUT],
        out_shape=[jax.ShapeDtypeStruct((B, H), f32),
                   jax.ShapeDtypeStruct((NBLK, 1, H), f32),
                   jax.ShapeDtypeStruct((NBLK, 1, H), f32)],
    )(se, ne, w1a, w1b, b1r)

    y, ps2, psq2 = pl.pallas_call(
        _mlp2_body,
        grid=(NBLK,),
        in_specs=[_blk((BBLK, H)), _PSUM_IN, _PSUM_IN,
                  _row((1, H)), _row((1, H)), _row((H, H)), _row((1, H))],
        out_specs=[_blk((BBLK, H)), _PSUM_OUT, _PSUM_OUT],
        out_shape=[jax.ShapeDtypeStruct((B, H), f32),
                   jax.ShapeDtypeStruct((NBLK, 1, H), f32),
                   jax.ShapeDtypeStruct((NBLK, 1, H), f32)],
    )(x1, ps, psq, g1r, be1r, W2, b2r)

    out = pl.pallas_call(
        _mlp3_body,
        grid=(NBLK,),
        in_specs=[_blk((BBLK, H)), _PSUM_IN, _PSUM_IN,
                  _row((1, H)), _row((1, H)), _row((H, H)), _row((1, H)),
                  _row((1, H)), _row((1, 1))],
        out_specs=_blk((BBLK, 1)),
        out_shape=jax.ShapeDtypeStruct((B, 1), f32),
    )(y, ps2, psq2, g2r, be2r, W3, b3r, w4r, b4r)

    return out.reshape(B)


def kernel(source, node, emb, W1, b1, g1, be1, W2, b2, g2, be2, W3, b3, W4, b4):
    emb_rm = _transpose_table(jnp.transpose(emb))
    se, ne = _sc_gather_mean(source, node, emb_rm)
    se, ne = _sc_gather_mean(source, node, emb_rm)
    return _mlp(se, ne, W1, b1, g1, be1, W2, b2, g2, be2, W3, b3, W4, b4)


# speedup vs baseline: 2.1990x; 1.9131x over previous
import jax
import jax.numpy as jnp
from jax import lax
from jax.experimental import pallas as pl
from jax.experimental.pallas import tpu as pltpu
from jax.experimental.pallas import tpu_sc as plsc

B = 16384
HIST = 50
D = 200
DP = 208
DT = 256
H = 256
NN = 1000000
NW = 32
BPW = B // NW
OCH = 32
NCH = 64
LANES = 16
NCHUNK = DP // LANES

RPG = 2
NSTEP = BPW // RPG
NBUF = 2


def _sc_body(src_hbm, node_hbm, emb_hbm, se_hbm, ne_hbm,
             idx_all, buf0, buf1, out_v, idxn, nbuf,
             sem0, sem1, semn):
    wid = lax.axis_index("s") * 2 + lax.axis_index("c")
    wbase = pl.multiple_of(wid * BPW, BPW)

    pltpu.sync_copy(src_hbm.at[pl.ds(wid * NSTEP, NSTEP), :], idx_all)

    pltpu.sync_copy(node_hbm.at[pl.ds(wbase, BPW)], idxn)

    def node_chunk(g, c):
        off = pl.multiple_of(g * NCH, NCH)
        pltpu.async_copy(emb_hbm.at[idxn.at[pl.ds(off, NCH)]], nbuf, semn).wait()
        pltpu.sync_copy(nbuf, ne_hbm.at[pl.ds(pl.multiple_of(wbase + off, NCH), NCH), :])
        return c

    lax.fori_loop(0, BPW // NCH, node_chunk, 0)

    bufs = (buf0, buf1)
    sems = (sem0, sem1)

    def start(t, k):
        pltpu.async_copy(emb_hbm.at[idx_all.at[t]], bufs[k], sems[k])

    def wait(t, k):
        pltpu.make_async_copy(emb_hbm.at[idx_all.at[t]], bufs[k], sems[k]).wait()

    def reduce_rows(t, k):
        buf = bufs[k]
        r = lax.rem(t * RPG, OCH)
        for p in range(RPG):
            def red(j, accs, p=p):
                return tuple(
                    accs[c] + buf[p * HIST + j, pl.ds(c * LANES, LANES)]
                    for c in range(NCHUNK)
                )

            accs = lax.fori_loop(
                0, HIST, red,
                tuple(jnp.zeros((LANES,), jnp.float32) for _ in range(NCHUNK)),
            )
            for c in range(NCHUNK):
                out_v[r + p, pl.ds(c * LANES, LANES)] = accs[c]

    for k in range(NBUF):
        start(k, k)

    def step_k(k):
        def go(t, c):
            wait(t, k)
            reduce_rows(t, k)

            @pl.when(t + NBUF < NSTEP)
            def _():
                start(t + NBUF, k)

            @pl.when(lax.rem(t * RPG, OCH) == OCH - RPG)
            def _():
                off = pl.multiple_of(wbase + t * RPG - (OCH - RPG), OCH)
                pltpu.sync_copy(out_v, se_hbm.at[pl.ds(off, OCH), :])

            return c
        return go

    def steps(g, c):
        t0 = g * NBUF
        for k in range(NBUF):
            c = step_k(k)(t0 + k, c)
        return c

    main = (NSTEP // NBUF) * NBUF
    lax.fori_loop(0, NSTEP // NBUF, steps, 0)
    for k in range(NSTEP - main):
        step_k(k)(main + k, 0)


def _sc_gather_mean(source, node, emb):
    mesh = plsc.VectorSubcoreMesh(core_axis_name="c", subcore_axis_name="s")
    f32 = jnp.float32
    run = pl.kernel(
        _sc_body,
        out_type=(
            jax.ShapeDtypeStruct((B, DP), f32),
            jax.ShapeDtypeStruct((B, DT), f32),
        ),
        mesh=mesh,
        scratch_types=[
            pltpu.VMEM((NSTEP, RPG * HIST), jnp.int32),
            pltpu.VMEM((RPG * HIST, DT), f32),
            pltpu.VMEM((RPG * HIST, DT), f32),
            pltpu.VMEM((OCH, DP), f32),
            pltpu.VMEM((BPW,), jnp.int32),
            pltpu.VMEM((NCH, DT), f32),
            pltpu.SemaphoreType.DMA,
            pltpu.SemaphoreType.DMA,
            pltpu.SemaphoreType.DMA,
        ],
    )
    return run(source.reshape(B // RPG, RPG * HIST), node, emb)


TBLK = 2048


def _transpose_body(src, dst):
    dst[:, pl.ds(0, D)] = src[...].T
    dst[:, pl.ds(D, DT - D)] = jnp.zeros((TBLK, DT - D), jnp.float32)


def _transpose_table(embT):
    return pl.pallas_call(
        _transpose_body,
        grid=(pl.cdiv(NN, TBLK),),
        in_specs=[pl.BlockSpec((D, TBLK), lambda i: (0, i))],
        out_specs=pl.BlockSpec((TBLK, DT), lambda i: (i, 0)),
        out_shape=jax.ShapeDtypeStruct((NN, DT), jnp.float32),
    )(embT)


BBLK = 2048
NBLK = B // BBLK


def _mlp1_body(se, ne, w1a, w1b, b1, x1, ps, psq):
    x = jnp.dot(se[...], w1a[...], preferred_element_type=jnp.float32)
    x = x + jnp.dot(ne[...], w1b[...], preferred_element_type=jnp.float32)
    x = jnp.maximum(x + b1[...], 0.0)
    x1[...] = x
    ps[...] = jnp.sum(x, axis=0, keepdims=True).reshape(1, 1, H)
    psq[...] = jnp.sum(x * x, axis=0, keepdims=True).reshape(1, 1, H)


def _mlp2_body(x1, ps, psq, g1, be1, w2, b2, y, ps2, psq2):
    m = jnp.sum(ps[...].reshape(NBLK, H), axis=0, keepdims=True) * (1.0 / B)
    ex2 = jnp.sum(psq[...].reshape(NBLK, H), axis=0, keepdims=True) * (1.0 / B)
    inv = lax.rsqrt(ex2 - m * m + 1e-5)
    x = (x1[...] - m) * (inv * g1[...]) + be1[...]
    x = jnp.maximum(jnp.dot(x, w2[...], preferred_element_type=jnp.float32) + b2[...], 0.0)
    y[...] = x
    ps2[...] = jnp.sum(x, axis=0, keepdims=True).reshape(1, 1, H)
    psq2[...] = jnp.sum(x * x, axis=0, keepdims=True).reshape(1, 1, H)


def _mlp3_body(y, ps2, psq2, g2, be2, w3, b3, w4r, b4, out):
    m = jnp.sum(ps2[...].reshape(NBLK, H), axis=0, keepdims=True) * (1.0 / B)
    ex2 = jnp.sum(psq2[...].reshape(NBLK, H), axis=0, keepdims=True) * (1.0 / B)
    inv = lax.rsqrt(ex2 - m * m + 1e-5)
    x = (y[...] - m) * (inv * g2[...]) + be2[...]
    x = jnp.maximum(jnp.dot(x, w3[...], preferred_element_type=jnp.float32) + b3[...], 0.0)
    o = jnp.sum(x * w4r[...], axis=1, keepdims=True) + b4[...]
    out[...] = 1.0 / (1.0 + jnp.exp(-o))


def _row(shape):
    return pl.BlockSpec(shape, lambda i: (0, 0))


def _blk(shape):
    return pl.BlockSpec(shape, lambda i: (i, 0))


_PSUM_OUT = pl.BlockSpec((1, 1, H), lambda i: (i, 0, 0))
_PSUM_IN = pl.BlockSpec((NBLK, 1, H), lambda i: (0, 0, 0))


def _mlp(se, ne, W1, b1, g1, be1, W2, b2, g2, be2, W3, b3, W4, b4):
    f32 = jnp.float32
    w1a = jnp.zeros((DP, H), f32).at[:D].set(W1[:D] * (1.0 / HIST))
    w1b = jnp.zeros((DT, H), f32).at[:D].set(W1[D:])
    b1r = b1.reshape(1, H)
    g1r = g1.reshape(1, H)
    be1r = be1.reshape(1, H)
    b2r = b2.reshape(1, H)
    g2r = g2.reshape(1, H)
    be2r = be2.reshape(1, H)
    b3r = b3.reshape(1, H)
    w4r = W4.reshape(1, H)
    b4r = b4.reshape(1, 1)

    x1, ps, psq = pl.pallas_call(
        _mlp1_body,
        grid=(NBLK,),
        in_specs=[_blk((BBLK, DP)), _blk((BBLK, DT)), _row((DP, H)), _row((DT, H)),
                  _row((1, H))],
        out_specs=[_blk((BBLK, H)), _PSUM_OUT, _PSUM_OUT],
        out_shape=[jax.ShapeDtypeStruct((B, H), f32),
                   jax.ShapeDtypeStruct((NBLK, 1, H), f32),
                   jax.ShapeDtypeStruct((NBLK, 1, H), f32)],
    )(se, ne, w1a, w1b, b1r)

    y, ps2, psq2 = pl.pallas_call(
        _mlp2_body,
        grid=(NBLK,),
        in_specs=[_blk((BBLK, H)), _PSUM_IN, _PSUM_IN,
                  _row((1, H)), _row((1, H)), _row((H, H)), _row((1, H))],
        out_specs=[_blk((BBLK, H)), _PSUM_OUT, _PSUM_OUT],
        out_shape=[jax.ShapeDtypeStruct((B, H), f32),
                   jax.ShapeDtypeStruct((NBLK, 1, H), f32),
                   jax.ShapeDtypeStruct((NBLK, 1, H), f32)],
    )(x1, ps, psq, g1r, be1r, W2, b2r)

    out = pl.pallas_call(
        _mlp3_body,
        grid=(NBLK,),
        in_specs=[_blk((BBLK, H)), _PSUM_IN, _PSUM_IN,
                  _row((1, H)), _row((1, H)), _row((H, H)), _row((1, H)),
                  _row((1, H)), _row((1, 1))],
        out_specs=_blk((BBLK, 1)),
        out_shape=jax.ShapeDtypeStruct((B, 1), f32),
    )(y, ps2, psq2, g2r, be2r, W3, b3r, w4r, b4r)

    return out.reshape(B)


def kernel(source, node, emb, W1, b1, g1, be1, W2, b2, g2, be2, W3, b3, W4, b4):
    emb_rm = _transpose_table(jnp.transpose(emb))
    se, ne = _sc_gather_mean(source, node, emb_rm)
    return _mlp(se, ne, W1, b1, g1, be1, W2, b2, g2, be2, W3, b3, W4, b4)


# speedup vs baseline: 2.4716x; 1.1240x over previous
import jax
import jax.numpy as jnp
from jax import lax
from jax.experimental import pallas as pl
from jax.experimental.pallas import tpu as pltpu
from jax.experimental.pallas import tpu_sc as plsc

B = 16384
HIST = 50
D = 200
DP = 208
DT = 256
H = 256
NN = 1000000
NW = 32
BPW = B // NW
OCH = 32
NCH = 64
LANES = 16
NCHUNK = DP // LANES

RPG = 2
NSTEP = BPW // RPG
NBUF = 2


def _sc_body(src_hbm, node_hbm, emb_hbm, se_hbm, ne_hbm,
             idx_all, buf0, buf1, out_v, idxn, nbuf,
             sem0, sem1, semn):
    wid = lax.axis_index("s") * 2 + lax.axis_index("c")
    wbase = pl.multiple_of(wid * BPW, BPW)

    pltpu.sync_copy(src_hbm.at[pl.ds(wid * NSTEP, NSTEP), :], idx_all)

    pltpu.sync_copy(node_hbm.at[pl.ds(wbase, BPW)], idxn)

    def node_chunk(g, c):
        off = pl.multiple_of(g * NCH, NCH)
        pltpu.async_copy(emb_hbm.at[idxn.at[pl.ds(off, NCH)]], nbuf, semn).wait()
        pltpu.sync_copy(nbuf, ne_hbm.at[pl.ds(pl.multiple_of(wbase + off, NCH), NCH), :])
        return c

    lax.fori_loop(0, BPW // NCH, node_chunk, 0)

    bufs = (buf0, buf1)
    sems = (sem0, sem1)

    def start(t, k):
        pltpu.async_copy(emb_hbm.at[idx_all.at[t]], bufs[k], sems[k])

    def wait(t, k):
        pltpu.make_async_copy(emb_hbm.at[idx_all.at[t]], bufs[k], sems[k]).wait()

    def reduce_rows(t, k):
        buf = bufs[k]
        r = lax.rem(t * RPG, OCH)
        for p in range(RPG):
            def red(j, accs, p=p):
                return tuple(
                    accs[c] + buf[p * HIST + j, pl.ds(c * LANES, LANES)]
                    for c in range(NCHUNK)
                )

            accs = lax.fori_loop(
                0, HIST, red,
                tuple(jnp.zeros((LANES,), jnp.float32) for _ in range(NCHUNK)),
            )
            for c in range(NCHUNK):
                out_v[r + p, pl.ds(c * LANES, LANES)] = accs[c]

    for k in range(NBUF):
        start(k, k)

    def step_k(k):
        def go(t, c):
            wait(t, k)
            reduce_rows(t, k)

            @pl.when(t + NBUF < NSTEP)
            def _():
                start(t + NBUF, k)

            @pl.when(lax.rem(t * RPG, OCH) == OCH - RPG)
            def _():
                off = pl.multiple_of(wbase + t * RPG - (OCH - RPG), OCH)
                pltpu.sync_copy(out_v, se_hbm.at[pl.ds(off, OCH), :])

            return c
        return go

    def steps(g, c):
        t0 = g * NBUF
        for k in range(NBUF):
            c = step_k(k)(t0 + k, c)
        return c

    main = (NSTEP // NBUF) * NBUF
    lax.fori_loop(0, NSTEP // NBUF, steps, 0)
    for k in range(NSTEP - main):
        step_k(k)(main + k, 0)


def _sc_gather_mean(source, node, emb):
    mesh = plsc.VectorSubcoreMesh(core_axis_name="c", subcore_axis_name="s")
    f32 = jnp.float32
    run = pl.kernel(
        _sc_body,
        out_type=(
            jax.ShapeDtypeStruct((B, DP), f32),
            jax.ShapeDtypeStruct((B, DT), f32),
        ),
        mesh=mesh,
        scratch_types=[
            pltpu.VMEM((NSTEP, RPG * HIST), jnp.int32),
            pltpu.VMEM((RPG * HIST, DT), f32),
            pltpu.VMEM((RPG * HIST, DT), f32),
            pltpu.VMEM((OCH, DP), f32),
            pltpu.VMEM((BPW,), jnp.int32),
            pltpu.VMEM((NCH, DT), f32),
            pltpu.SemaphoreType.DMA,
            pltpu.SemaphoreType.DMA,
            pltpu.SemaphoreType.DMA,
        ],
    )
    return run(source.reshape(B // RPG, RPG * HIST), node, emb)


TBLK = 4096


def _transpose_body(src, dst):
    dst[:, pl.ds(0, D)] = src[...].T
    dst[:, pl.ds(D, DT - D)] = jnp.zeros((TBLK, DT - D), jnp.float32)


def _transpose_table(embT):
    return pl.pallas_call(
        _transpose_body,
        grid=(pl.cdiv(NN, TBLK),),
        in_specs=[pl.BlockSpec((D, TBLK), lambda i: (0, i))],
        out_specs=pl.BlockSpec((TBLK, DT), lambda i: (i, 0)),
        out_shape=jax.ShapeDtypeStruct((NN, DT), jnp.float32),
    )(embT)


BBLK = 2048
NBLK = B // BBLK


def _mlp1_body(se, ne, w1a, w1b, b1, x1, ps, psq):
    x = jnp.dot(se[...], w1a[...], preferred_element_type=jnp.float32)
    x = x + jnp.dot(ne[...], w1b[...], preferred_element_type=jnp.float32)
    x = jnp.maximum(x + b1[...], 0.0)
    x1[...] = x
    ps[...] = jnp.sum(x, axis=0, keepdims=True).reshape(1, 1, H)
    psq[...] = jnp.sum(x * x, axis=0, keepdims=True).reshape(1, 1, H)


def _mlp2_body(x1, ps, psq, g1, be1, w2, b2, y, ps2, psq2):
    m = jnp.sum(ps[...].reshape(NBLK, H), axis=0, keepdims=True) * (1.0 / B)
    ex2 = jnp.sum(psq[...].reshape(NBLK, H), axis=0, keepdims=True) * (1.0 / B)
    inv = lax.rsqrt(ex2 - m * m + 1e-5)
    x = (x1[...] - m) * (inv * g1[...]) + be1[...]
    x = jnp.maximum(jnp.dot(x, w2[...], preferred_element_type=jnp.float32) + b2[...], 0.0)
    y[...] = x
    ps2[...] = jnp.sum(x, axis=0, keepdims=True).reshape(1, 1, H)
    psq2[...] = jnp.sum(x * x, axis=0, keepdims=True).reshape(1, 1, H)


def _mlp3_body(y, ps2, psq2, g2, be2, w3, b3, w4r, b4, out):
    m = jnp.sum(ps2[...].reshape(NBLK, H), axis=0, keepdims=True) * (1.0 / B)
    ex2 = jnp.sum(psq2[...].reshape(NBLK, H), axis=0, keepdims=True) * (1.0 / B)
    inv = lax.rsqrt(ex2 - m * m + 1e-5)
    x = (y[...] - m) * (inv * g2[...]) + be2[...]
    x = jnp.maximum(jnp.dot(x, w3[...], preferred_element_type=jnp.float32) + b3[...], 0.0)
    o = jnp.sum(x * w4r[...], axis=1, keepdims=True) + b4[...]
    out[...] = 1.0 / (1.0 + jnp.exp(-o))


def _row(shape):
    return pl.BlockSpec(shape, lambda i: (0, 0))


def _blk(shape):
    return pl.BlockSpec(shape, lambda i: (i, 0))


_PSUM_OUT = pl.BlockSpec((1, 1, H), lambda i: (i, 0, 0))
_PSUM_IN = pl.BlockSpec((NBLK, 1, H), lambda i: (0, 0, 0))


def _mlp(se, ne, W1, b1, g1, be1, W2, b2, g2, be2, W3, b3, W4, b4):
    f32 = jnp.float32
    w1a = jnp.zeros((DP, H), f32).at[:D].set(W1[:D] * (1.0 / HIST))
    w1b = jnp.zeros((DT, H), f32).at[:D].set(W1[D:])
    b1r = b1.reshape(1, H)
    g1r = g1.reshape(1, H)
    be1r = be1.reshape(1, H)
    b2r = b2.reshape(1, H)
    g2r = g2.reshape(1, H)
    be2r = be2.reshape(1, H)
    b3r = b3.reshape(1, H)
    w4r = W4.reshape(1, H)
    b4r = b4.reshape(1, 1)

    x1, ps, psq = pl.pallas_call(
        _mlp1_body,
        grid=(NBLK,),
        in_specs=[_blk((BBLK, DP)), _blk((BBLK, DT)), _row((DP, H)), _row((DT, H)),
                  _row((1, H))],
        out_specs=[_blk((BBLK, H)), _PSUM_OUT, _PSUM_OUT],
        out_shape=[jax.ShapeDtypeStruct((B, H), f32),
                   jax.ShapeDtypeStruct((NBLK, 1, H), f32),
                   jax.ShapeDtypeStruct((NBLK, 1, H), f32)],
    )(se, ne, w1a, w1b, b1r)

    y, ps2, psq2 = pl.pallas_call(
        _mlp2_body,
        grid=(NBLK,),
        in_specs=[_blk((BBLK, H)), _PSUM_IN, _PSUM_IN,
                  _row((1, H)), _row((1, H)), _row((H, H)), _row((1, H))],
        out_specs=[_blk((BBLK, H)), _PSUM_OUT, _PSUM_OUT],
        out_shape=[jax.ShapeDtypeStruct((B, H), f32),
                   jax.ShapeDtypeStruct((NBLK, 1, H), f32),
                   jax.ShapeDtypeStruct((NBLK, 1, H), f32)],
    )(x1, ps, psq, g1r, be1r, W2, b2r)

    out = pl.pallas_call(
        _mlp3_body,
        grid=(NBLK,),
        in_specs=[_blk((BBLK, H)), _PSUM_IN, _PSUM_IN,
                  _row((1, H)), _row((1, H)), _row((H, H)), _row((1, H)),
                  _row((1, H)), _row((1, 1))],
        out_specs=_blk((BBLK, 1)),
        out_shape=jax.ShapeDtypeStruct((B, 1), f32),
    )(y, ps2, psq2, g2r, be2r, W3, b3r, w4r, b4r)

    return out.reshape(B)


def kernel(source, node, emb, W1, b1, g1, be1, W2, b2, g2, be2, W3, b3, W4, b4):
    emb_rm = _transpose_table(jnp.transpose(emb))
    se, ne = _sc_gather_mean(source, node, emb_rm)
    return _mlp(se, ne, W1, b1, g1, be1, W2, b2, g2, be2, W3, b3, W4, b4)


# speedup vs baseline: 2.9332x; 1.1868x over previous
import jax
import jax.numpy as jnp
from jax import lax
from jax.experimental import pallas as pl
from jax.experimental.pallas import tpu as pltpu
from jax.experimental.pallas import tpu_sc as plsc

B = 16384
HIST = 50
D = 200
DH = D // 2
DT = 128
PW = 224
H = 256
NN = 1000000
NW = 32
BPW = B // NW
OCH = 32
NCH = 128
LANES = 16
NCHUNK = 7

RPG = 2
NSTEP = BPW // RPG
NBUF = 3


def _sc_body(src_hbm, node_hbm, emb_hbm, se_hbm, ne_hbm,
             idx_all, buf0, buf1, buf2, out_v, idxn, nbuf,
             sem0, sem1, sem2, semn):
    wid = lax.axis_index("s") * 2 + lax.axis_index("c")
    wbase = pl.multiple_of(wid * BPW, BPW)

    pltpu.sync_copy(src_hbm.at[pl.ds(wid * NSTEP, NSTEP), :], idx_all)

    pltpu.sync_copy(node_hbm.at[pl.ds(wbase, BPW)], idxn)

    def node_chunk(g, c):
        off = pl.multiple_of(g * NCH, NCH)
        pltpu.async_copy(emb_hbm.at[idxn.at[pl.ds(off, NCH)]], nbuf, semn).wait()
        pltpu.sync_copy(nbuf, ne_hbm.at[pl.ds(pl.multiple_of(wbase + off, NCH), NCH), :])
        return c

    lax.fori_loop(0, BPW // NCH, node_chunk, 0)

    bufs = (buf0, buf1, buf2)
    sems = (sem0, sem1, sem2)

    def start(t, k):
        pltpu.async_copy(emb_hbm.at[idx_all.at[t]], bufs[k], sems[k])

    def wait(t, k):
        pltpu.make_async_copy(emb_hbm.at[idx_all.at[t]], bufs[k], sems[k]).wait()

    def reduce_rows(t, k):
        buf = bufs[k]
        r = lax.rem(t * RPG, OCH)
        for p in range(RPG):
            def red(j, accs, p=p):
                new = list(accs)
                for c in range(NCHUNK):
                    w = buf[p * HIST + j, pl.ds(c * LANES, LANES)]
                    lo, hi = plsc.unpack(
                        plsc.bitcast(w, jnp.bfloat16),
                        format=plsc.PackFormat.INTERLEAVED,
                    )
                    new[c] = new[c] + lo
                    new[NCHUNK + c] = new[NCHUNK + c] + hi
                return tuple(new)

            accs = lax.fori_loop(
                0, HIST, red,
                tuple(jnp.zeros((LANES,), jnp.float32) for _ in range(2 * NCHUNK)),
            )
            for c in range(2 * NCHUNK):
                out_v[r + p, pl.ds(c * LANES, LANES)] = accs[c]

    for k in range(NBUF):
        start(k, k)

    def step_k(k):
        def go(t, c):
            wait(t, k)
            reduce_rows(t, k)

            @pl.when(t + NBUF < NSTEP)
            def _():
                start(t + NBUF, k)

            @pl.when(lax.rem(t * RPG, OCH) == OCH - RPG)
            def _():
                off = pl.multiple_of(wbase + t * RPG - (OCH - RPG), OCH)
                pltpu.sync_copy(out_v, se_hbm.at[pl.ds(off, OCH), :])

            return c
        return go

    def steps(g, c):
        t0 = g * NBUF
        for k in range(NBUF):
            c = step_k(k)(t0 + k, c)
        return c

    main = (NSTEP // NBUF) * NBUF
    lax.fori_loop(0, NSTEP // NBUF, steps, 0)
    for k in range(NSTEP - main):
        step_k(k)(main + k, 0)


def _sc_gather_mean(source, node, emb):
    mesh = plsc.VectorSubcoreMesh(core_axis_name="c", subcore_axis_name="s")
    f32 = jnp.float32
    run = pl.kernel(
        _sc_body,
        out_type=(
            jax.ShapeDtypeStruct((B, PW), f32),
            jax.ShapeDtypeStruct((B, DT), f32),
        ),
        mesh=mesh,
        scratch_types=[
            pltpu.VMEM((NSTEP, RPG * HIST), jnp.int32),
            pltpu.VMEM((RPG * HIST, DT), f32),
            pltpu.VMEM((RPG * HIST, DT), f32),
            pltpu.VMEM((RPG * HIST, DT), f32),
            pltpu.VMEM((OCH, PW), f32),
            pltpu.VMEM((BPW,), jnp.int32),
            pltpu.VMEM((NCH, DT), f32),
            pltpu.SemaphoreType.DMA,
            pltpu.SemaphoreType.DMA,
            pltpu.SemaphoreType.DMA,
            pltpu.SemaphoreType.DMA,
        ],
        compiler_params=pltpu.CompilerParams(needs_layout_passes=False),
    )
    return run(source.reshape(B // RPG, RPG * HIST), node, emb)


TBLK = 4096


def _transpose_body(src, dst):
    x = src[...].T
    xb = x.astype(jnp.bfloat16)
    lo = lax.convert_element_type(
        lax.bitcast_convert_type(xb[:, :DH], jnp.uint16), jnp.uint32)
    hi = lax.convert_element_type(
        lax.bitcast_convert_type(xb[:, DH:], jnp.uint16), jnp.uint32)
    dst[:, pl.ds(0, DH)] = lax.bitcast_convert_type((hi << 16) | lo, jnp.float32)
    dst[:, pl.ds(DH, DT - DH)] = jnp.zeros((TBLK, DT - DH), jnp.float32)


def _transpose_table(embT):
    return pl.pallas_call(
        _transpose_body,
        grid=(pl.cdiv(NN, TBLK),),
        in_specs=[pl.BlockSpec((D, TBLK), lambda i: (0, i))],
        out_specs=pl.BlockSpec((TBLK, DT), lambda i: (i, 0)),
        out_shape=jax.ShapeDtypeStruct((NN, DT), jnp.float32),
    )(embT)


BBLK = 2048
NBLK = B // BBLK


def _mlp1_body(se, ne, w1a, w1bl, w1bh, b1, x1, ps, psq):
    wu = lax.bitcast_convert_type(ne[...], jnp.uint32)
    ne_lo = lax.bitcast_convert_type(wu << 16, jnp.float32)
    ne_hi = lax.bitcast_convert_type(wu & jnp.uint32(0xFFFF0000), jnp.float32)
    x = jnp.dot(se[...], w1a[...], preferred_element_type=jnp.float32)
    x = x + jnp.dot(ne_lo, w1bl[...], preferred_element_type=jnp.float32)
    x = x + jnp.dot(ne_hi, w1bh[...], preferred_element_type=jnp.float32)
    x = jnp.maximum(x + b1[...], 0.0)
    x1[...] = x
    ps[...] = jnp.sum(x, axis=0, keepdims=True).reshape(1, 1, H)
    psq[...] = jnp.sum(x * x, axis=0, keepdims=True).reshape(1, 1, H)


def _mlp2_body(x1, ps, psq, g1, be1, w2, b2, y, ps2, psq2):
    m = jnp.sum(ps[...].reshape(NBLK, H), axis=0, keepdims=True) * (1.0 / B)
    ex2 = jnp.sum(psq[...].reshape(NBLK, H), axis=0, keepdims=True) * (1.0 / B)
    inv = lax.rsqrt(ex2 - m * m + 1e-5)
    x = (x1[...] - m) * (inv * g1[...]) + be1[...]
    x = jnp.maximum(jnp.dot(x, w2[...], preferred_element_type=jnp.float32) + b2[...], 0.0)
    y[...] = x
    ps2[...] = jnp.sum(x, axis=0, keepdims=True).reshape(1, 1, H)
    psq2[...] = jnp.sum(x * x, axis=0, keepdims=True).reshape(1, 1, H)


def _mlp3_body(y, ps2, psq2, g2, be2, w3, b3, w4r, b4, out):
    m = jnp.sum(ps2[...].reshape(NBLK, H), axis=0, keepdims=True) * (1.0 / B)
    ex2 = jnp.sum(psq2[...].reshape(NBLK, H), axis=0, keepdims=True) * (1.0 / B)
    inv = lax.rsqrt(ex2 - m * m + 1e-5)
    x = (y[...] - m) * (inv * g2[...]) + be2[...]
    x = jnp.maximum(jnp.dot(x, w3[...], preferred_element_type=jnp.float32) + b3[...], 0.0)
    o = jnp.sum(x * w4r[...], axis=1, keepdims=True) + b4[...]
    out[...] = 1.0 / (1.0 + jnp.exp(-o))


def _row(shape):
    return pl.BlockSpec(shape, lambda i: (0, 0))


def _blk(shape):
    return pl.BlockSpec(shape, lambda i: (i, 0))


_PSUM_OUT = pl.BlockSpec((1, 1, H), lambda i: (i, 0, 0))
_PSUM_IN = pl.BlockSpec((NBLK, 1, H), lambda i: (0, 0, 0))


def _mlp(se, ne, W1, b1, g1, be1, W2, b2, g2, be2, W3, b3, W4, b4):
    f32 = jnp.float32
    w1s = W1[:D] * (1.0 / HIST)
    w1a = (jnp.zeros((PW, H), f32)
           .at[:DH].set(w1s[:DH])
           .at[DH + 12:DH + 12 + DH].set(w1s[DH:]))
    w1bl = jnp.zeros((DT, H), f32).at[:DH].set(W1[D:D + DH])
    w1bh = jnp.zeros((DT, H), f32).at[:DH].set(W1[D + DH:])
    b1r = b1.reshape(1, H)
    g1r = g1.reshape(1, H)
    be1r = be1.reshape(1, H)
    b2r = b2.reshape(1, H)
    g2r = g2.reshape(1, H)
    be2r = be2.reshape(1, H)
    b3r = b3.reshape(1, H)
    w4r = W4.reshape(1, H)
    b4r = b4.reshape(1, 1)

    x1, ps, psq = pl.pallas_call(
        _mlp1_body,
        grid=(NBLK,),
        in_specs=[_blk((BBLK, PW)), _blk((BBLK, DT)), _row((PW, H)),
                  _row((DT, H)), _row((DT, H)), _row((1, H))],
        out_specs=[_blk((BBLK, H)), _PSUM_OUT, _PSUM_OUT],
        out_shape=[jax.ShapeDtypeStruct((B, H), f32),
                   jax.ShapeDtypeStruct((NBLK, 1, H), f32),
                   jax.ShapeDtypeStruct((NBLK, 1, H), f32)],
    )(se, ne, w1a, w1bl, w1bh, b1r)

    y, ps2, psq2 = pl.pallas_call(
        _mlp2_body,
        grid=(NBLK,),
        in_specs=[_blk((BBLK, H)), _PSUM_IN, _PSUM_IN,
                  _row((1, H)), _row((1, H)), _row((H, H)), _row((1, H))],
        out_specs=[_blk((BBLK, H)), _PSUM_OUT, _PSUM_OUT],
        out_shape=[jax.ShapeDtypeStruct((B, H), f32),
                   jax.ShapeDtypeStruct((NBLK, 1, H), f32),
                   jax.ShapeDtypeStruct((NBLK, 1, H), f32)],
    )(x1, ps, psq, g1r, be1r, W2, b2r)

    out = pl.pallas_call(
        _mlp3_body,
        grid=(NBLK,),
        in_specs=[_blk((BBLK, H)), _PSUM_IN, _PSUM_IN,
                  _row((1, H)), _row((1, H)), _row((H, H)), _row((1, H)),
                  _row((1, H)), _row((1, 1))],
        out_specs=_blk((BBLK, 1)),
        out_shape=jax.ShapeDtypeStruct((B, 1), f32),
    )(y, ps2, psq2, g2r, be2r, W3, b3r, w4r, b4r)

    return out.reshape(B)


def kernel(source, node, emb, W1, b1, g1, be1, W2, b2, g2, be2, W3, b3, W4, b4):
    emb_rm = _transpose_table(jnp.transpose(emb))
    se, ne = _sc_gather_mean(source, node, emb_rm)
    return _mlp(se, ne, W1, b1, g1, be1, W2, b2, g2, be2, W3, b3, W4, b4)


# speedup vs baseline: 3.5924x; 1.2248x over previous
import jax
import jax.numpy as jnp
from jax import lax
from jax.experimental import pallas as pl
from jax.experimental.pallas import tpu as pltpu
from jax.experimental.pallas import tpu_sc as plsc

B = 16384
HIST = 50
D = 200
DH = D // 2
DT = 128
PW = 224
H = 256
NN = 1000000
NW = 32
BPW = B // NW
OCH = 32
NCH = 128
LANES = 16
NCHUNK = 7

RPG = 2
NSTEP = BPW // RPG
NBUF = 3


def _sc_body(src_hbm, node_hbm, emb_hbm, se_hbm, ne_hbm,
             idx_all, buf0, buf1, buf2, out_v, idxn, nbuf,
             sem0, sem1, sem2, semn):
    wid = lax.axis_index("s") * 2 + lax.axis_index("c")
    wbase = pl.multiple_of(wid * BPW, BPW)

    pltpu.sync_copy(src_hbm.at[pl.ds(wid * NSTEP, NSTEP), :], idx_all)

    pltpu.sync_copy(node_hbm.at[pl.ds(wbase, BPW)], idxn)

    def node_chunk(g, c):
        off = pl.multiple_of(g * NCH, NCH)
        pltpu.async_copy(emb_hbm.at[idxn.at[pl.ds(off, NCH)]], nbuf, semn).wait()
        pltpu.sync_copy(nbuf, ne_hbm.at[pl.ds(pl.multiple_of(wbase + off, NCH), NCH), :])
        return c

    lax.fori_loop(0, BPW // NCH, node_chunk, 0)

    bufs = (buf0, buf1, buf2)
    sems = (sem0, sem1, sem2)

    def start(t, k):
        pltpu.async_copy(emb_hbm.at[idx_all.at[t]], bufs[k], sems[k])

    def wait(t, k):
        pltpu.make_async_copy(emb_hbm.at[idx_all.at[t]], bufs[k], sems[k]).wait()

    def reduce_rows(t, k):
        buf = bufs[k]
        r = lax.rem(t * RPG, OCH)
        for p in range(RPG):
            def red(j, accs, p=p):
                new = list(accs)
                for c in range(NCHUNK):
                    w = buf[p * HIST + j, pl.ds(c * LANES, LANES)]
                    lo, hi = plsc.unpack(
                        plsc.bitcast(w, jnp.bfloat16),
                        format=plsc.PackFormat.INTERLEAVED,
                    )
                    new[c] = new[c] + lo
                    new[NCHUNK + c] = new[NCHUNK + c] + hi
                return tuple(new)

            accs = lax.fori_loop(
                0, HIST, red,
                tuple(jnp.zeros((LANES,), jnp.float32) for _ in range(2 * NCHUNK)),
            )
            for c in range(2 * NCHUNK):
                out_v[r + p, pl.ds(c * LANES, LANES)] = accs[c]

    for k in range(NBUF):
        start(k, k)

    def step_k(k):
        def go(t, c):
            wait(t, k)
            reduce_rows(t, k)

            @pl.when(t + NBUF < NSTEP)
            def _():
                start(t + NBUF, k)

            @pl.when(lax.rem(t * RPG, OCH) == OCH - RPG)
            def _():
                off = pl.multiple_of(wbase + t * RPG - (OCH - RPG), OCH)
                pltpu.sync_copy(out_v, se_hbm.at[pl.ds(off, OCH), :])

            return c
        return go

    def steps(g, c):
        t0 = g * NBUF
        for k in range(NBUF):
            c = step_k(k)(t0 + k, c)
        return c

    main = (NSTEP // NBUF) * NBUF
    lax.fori_loop(0, NSTEP // NBUF, steps, 0)
    for k in range(NSTEP - main):
        step_k(k)(main + k, 0)


def _sc_gather_mean(source, node, emb):
    mesh = plsc.VectorSubcoreMesh(core_axis_name="c", subcore_axis_name="s")
    f32 = jnp.float32
    run = pl.kernel(
        _sc_body,
        out_type=(
            jax.ShapeDtypeStruct((B, PW), f32),
            jax.ShapeDtypeStruct((B, DT), f32),
        ),
        mesh=mesh,
        scratch_types=[
            pltpu.VMEM((NSTEP, RPG * HIST), jnp.int32),
            pltpu.VMEM((RPG * HIST, DT), f32),
            pltpu.VMEM((RPG * HIST, DT), f32),
            pltpu.VMEM((RPG * HIST, DT), f32),
            pltpu.VMEM((OCH, PW), f32),
            pltpu.VMEM((BPW,), jnp.int32),
            pltpu.VMEM((NCH, DT), f32),
            pltpu.SemaphoreType.DMA,
            pltpu.SemaphoreType.DMA,
            pltpu.SemaphoreType.DMA,
            pltpu.SemaphoreType.DMA,
        ],
        compiler_params=pltpu.CompilerParams(needs_layout_passes=False),
    )
    return run(source.reshape(B // RPG, RPG * HIST), node, emb)


TBLK = 4096


def _transpose_body(src, dst):
    x = src[...]
    lo = lax.convert_element_type(
        lax.bitcast_convert_type(x[:DH, :].astype(jnp.bfloat16), jnp.uint16),
        jnp.uint32)
    hi = lax.convert_element_type(
        lax.bitcast_convert_type(x[DH:, :].astype(jnp.bfloat16), jnp.uint16),
        jnp.uint32)
    w = lax.bitcast_convert_type((hi << 16) | lo, jnp.float32)
    dst[:, pl.ds(0, DH)] = w.T
    dst[:, pl.ds(DH, DT - DH)] = jnp.zeros((TBLK, DT - DH), jnp.float32)


def _transpose_table(embT):
    return pl.pallas_call(
        _transpose_body,
        grid=(pl.cdiv(NN, TBLK),),
        in_specs=[pl.BlockSpec((D, TBLK), lambda i: (0, i))],
        out_specs=pl.BlockSpec((TBLK, DT), lambda i: (i, 0)),
        out_shape=jax.ShapeDtypeStruct((NN, DT), jnp.float32),
    )(embT)


BBLK = 2048
NBLK = B // BBLK


def _mlp1_body(se, ne, w1a, w1bl, w1bh, b1, x1, ps, psq):
    wu = lax.bitcast_convert_type(ne[...], jnp.uint32)
    ne_lo = lax.bitcast_convert_type(wu << 16, jnp.float32)
    ne_hi = lax.bitcast_convert_type(wu & jnp.uint32(0xFFFF0000), jnp.float32)
    x = jnp.dot(se[...], w1a[...], preferred_element_type=jnp.float32)
    x = x + jnp.dot(ne_lo, w1bl[...], preferred_element_type=jnp.float32)
    x = x + jnp.dot(ne_hi, w1bh[...], preferred_element_type=jnp.float32)
    x = jnp.maximum(x + b1[...], 0.0)
    x1[...] = x
    ps[...] = jnp.sum(x, axis=0, keepdims=True).reshape(1, 1, H)
    psq[...] = jnp.sum(x * x, axis=0, keepdims=True).reshape(1, 1, H)


def _mlp2_body(x1, ps, psq, g1, be1, w2, b2, y, ps2, psq2):
    m = jnp.sum(ps[...].reshape(NBLK, H), axis=0, keepdims=True) * (1.0 / B)
    ex2 = jnp.sum(psq[...].reshape(NBLK, H), axis=0, keepdims=True) * (1.0 / B)
    inv = lax.rsqrt(ex2 - m * m + 1e-5)
    x = (x1[...] - m) * (inv * g1[...]) + be1[...]
    x = jnp.maximum(jnp.dot(x, w2[...], preferred_element_type=jnp.float32) + b2[...], 0.0)
    y[...] = x
    ps2[...] = jnp.sum(x, axis=0, keepdims=True).reshape(1, 1, H)
    psq2[...] = jnp.sum(x * x, axis=0, keepdims=True).reshape(1, 1, H)


def _mlp3_body(y, ps2, psq2, g2, be2, w3, b3, w4r, b4, out):
    m = jnp.sum(ps2[...].reshape(NBLK, H), axis=0, keepdims=True) * (1.0 / B)
    ex2 = jnp.sum(psq2[...].reshape(NBLK, H), axis=0, keepdims=True) * (1.0 / B)
    inv = lax.rsqrt(ex2 - m * m + 1e-5)
    x = (y[...] - m) * (inv * g2[...]) + be2[...]
    x = jnp.maximum(jnp.dot(x, w3[...], preferred_element_type=jnp.float32) + b3[...], 0.0)
    o = jnp.sum(x * w4r[...], axis=1, keepdims=True) + b4[...]
    out[...] = 1.0 / (1.0 + jnp.exp(-o))


def _row(shape):
    return pl.BlockSpec(shape, lambda i: (0, 0))


def _blk(shape):
    return pl.BlockSpec(shape, lambda i: (i, 0))


_PSUM_OUT = pl.BlockSpec((1, 1, H), lambda i: (i, 0, 0))
_PSUM_IN = pl.BlockSpec((NBLK, 1, H), lambda i: (0, 0, 0))


def _mlp(se, ne, W1, b1, g1, be1, W2, b2, g2, be2, W3, b3, W4, b4):
    f32 = jnp.float32
    w1s = W1[:D] * (1.0 / HIST)
    w1a = (jnp.zeros((PW, H), f32)
           .at[:DH].set(w1s[:DH])
           .at[DH + 12:DH + 12 + DH].set(w1s[DH:]))
    w1bl = jnp.zeros((DT, H), f32).at[:DH].set(W1[D:D + DH])
    w1bh = jnp.zeros((DT, H), f32).at[:DH].set(W1[D + DH:])
    b1r = b1.reshape(1, H)
    g1r = g1.reshape(1, H)
    be1r = be1.reshape(1, H)
    b2r = b2.reshape(1, H)
    g2r = g2.reshape(1, H)
    be2r = be2.reshape(1, H)
    b3r = b3.reshape(1, H)
    w4r = W4.reshape(1, H)
    b4r = b4.reshape(1, 1)

    x1, ps, psq = pl.pallas_call(
        _mlp1_body,
        grid=(NBLK,),
        in_specs=[_blk((BBLK, PW)), _blk((BBLK, DT)), _row((PW, H)),
                  _row((DT, H)), _row((DT, H)), _row((1, H))],
        out_specs=[_blk((BBLK, H)), _PSUM_OUT, _PSUM_OUT],
        out_shape=[jax.ShapeDtypeStruct((B, H), f32),
                   jax.ShapeDtypeStruct((NBLK, 1, H), f32),
                   jax.ShapeDtypeStruct((NBLK, 1, H), f32)],
    )(se, ne, w1a, w1bl, w1bh, b1r)

    y, ps2, psq2 = pl.pallas_call(
        _mlp2_body,
        grid=(NBLK,),
        in_specs=[_blk((BBLK, H)), _PSUM_IN, _PSUM_IN,
                  _row((1, H)), _row((1, H)), _row((H, H)), _row((1, H))],
        out_specs=[_blk((BBLK, H)), _PSUM_OUT, _PSUM_OUT],
        out_shape=[jax.ShapeDtypeStruct((B, H), f32),
                   jax.ShapeDtypeStruct((NBLK, 1, H), f32),
                   jax.ShapeDtypeStruct((NBLK, 1, H), f32)],
    )(x1, ps, psq, g1r, be1r, W2, b2r)

    out = pl.pallas_call(
        _mlp3_body,
        grid=(NBLK,),
        in_specs=[_blk((BBLK, H)), _PSUM_IN, _PSUM_IN,
                  _row((1, H)), _row((1, H)), _row((H, H)), _row((1, H)),
                  _row((1, H)), _row((1, 1))],
        out_specs=_blk((BBLK, 1)),
        out_shape=jax.ShapeDtypeStruct((B, 1), f32),
    )(y, ps2, psq2, g2r, be2r, W3, b3r, w4r, b4r)

    return out.reshape(B)


def kernel(source, node, emb, W1, b1, g1, be1, W2, b2, g2, be2, W3, b3, W4, b4):
    emb_rm = _transpose_table(jnp.transpose(emb))
    se, ne = _sc_gather_mean(source, node, emb_rm)
    return _mlp(se, ne, W1, b1, g1, be1, W2, b2, g2, be2, W3, b3, W4, b4)


# speedup vs baseline: 3.7711x; 1.0497x over previous
import jax
import jax.numpy as jnp
from jax import lax
from jax.experimental import pallas as pl
from jax.experimental.pallas import tpu as pltpu
from jax.experimental.pallas import tpu_sc as plsc

B = 16384
HIST = 50
D = 200
DH = D // 2
DT = 128
PW = 224
H = 256
NN = 1000000
NW = 32
BPW = B // NW
OCH = 32
NCH = 128
LANES = 16
NCHUNK = 7

RPG = 2
NSTEP = BPW // RPG
NBUF = 3


def _sc_body(src_hbm, node_hbm, emb_hbm, se_hbm, ne_hbm,
             idx_all, buf0, buf1, buf2, out_v, idxn, nbuf,
             sem0, sem1, sem2, semn):
    wid = lax.axis_index("s") * 2 + lax.axis_index("c")
    wbase = pl.multiple_of(wid * BPW, BPW)

    pltpu.sync_copy(src_hbm.at[pl.ds(wid * NSTEP, NSTEP), :], idx_all)

    pltpu.sync_copy(node_hbm.at[pl.ds(wbase, BPW)], idxn)

    def node_chunk(g, c):
        off = pl.multiple_of(g * NCH, NCH)
        pltpu.async_copy(emb_hbm.at[idxn.at[pl.ds(off, NCH)]], nbuf, semn).wait()
        pltpu.sync_copy(nbuf, ne_hbm.at[pl.ds(pl.multiple_of(wbase + off, NCH), NCH), :])
        return c

    lax.fori_loop(0, BPW // NCH, node_chunk, 0)

    bufs = (buf0, buf1, buf2)
    sems = (sem0, sem1, sem2)

    def start(t, k):
        pltpu.async_copy(emb_hbm.at[idx_all.at[t]], bufs[k], sems[k])

    def wait(t, k):
        pltpu.make_async_copy(emb_hbm.at[idx_all.at[t]], bufs[k], sems[k]).wait()

    def reduce_rows(t, k):
        buf = bufs[k]
        r = lax.rem(t * RPG, OCH)
        for p in range(RPG):
            def red(j, accs, p=p):
                new = list(accs)
                for c in range(NCHUNK):
                    w = buf[p * HIST + j, pl.ds(c * LANES, LANES)]
                    lo, hi = plsc.unpack(
                        plsc.bitcast(w, jnp.bfloat16),
                        format=plsc.PackFormat.INTERLEAVED,
                    )
                    new[c] = new[c] + lo
                    new[NCHUNK + c] = new[NCHUNK + c] + hi
                return tuple(new)

            accs = lax.fori_loop(
                0, HIST, red,
                tuple(jnp.zeros((LANES,), jnp.float32) for _ in range(2 * NCHUNK)),
            )
            for c in range(2 * NCHUNK):
                out_v[r + p, pl.ds(c * LANES, LANES)] = accs[c]

    for k in range(NBUF):
        start(k, k)

    def step_k(k):
        def go(t, c):
            wait(t, k)
            reduce_rows(t, k)

            @pl.when(t + NBUF < NSTEP)
            def _():
                start(t + NBUF, k)

            @pl.when(lax.rem(t * RPG, OCH) == OCH - RPG)
            def _():
                off = pl.multiple_of(wbase + t * RPG - (OCH - RPG), OCH)
                pltpu.sync_copy(out_v, se_hbm.at[pl.ds(off, OCH), :])

            return c
        return go

    def steps(g, c):
        t0 = g * NBUF
        for k in range(NBUF):
            c = step_k(k)(t0 + k, c)
        return c

    main = (NSTEP // NBUF) * NBUF
    lax.fori_loop(0, NSTEP // NBUF, steps, 0)
    for k in range(NSTEP - main):
        step_k(k)(main + k, 0)


def _sc_gather_mean(source, node, emb):
    mesh = plsc.VectorSubcoreMesh(core_axis_name="c", subcore_axis_name="s")
    f32 = jnp.float32
    run = pl.kernel(
        _sc_body,
        out_type=(
            jax.ShapeDtypeStruct((B, PW), f32),
            jax.ShapeDtypeStruct((B, DT), f32),
        ),
        mesh=mesh,
        scratch_types=[
            pltpu.VMEM((NSTEP, RPG * HIST), jnp.int32),
            pltpu.VMEM((RPG * HIST, DT), f32),
            pltpu.VMEM((RPG * HIST, DT), f32),
            pltpu.VMEM((RPG * HIST, DT), f32),
            pltpu.VMEM((OCH, PW), f32),
            pltpu.VMEM((BPW,), jnp.int32),
            pltpu.VMEM((NCH, DT), f32),
            pltpu.SemaphoreType.DMA,
            pltpu.SemaphoreType.DMA,
            pltpu.SemaphoreType.DMA,
            pltpu.SemaphoreType.DMA,
        ],
        compiler_params=pltpu.CompilerParams(needs_layout_passes=False),
    )
    return run(source.reshape(B // RPG, RPG * HIST), node, emb)


TBLK = 8192


def _transpose_body(src, dst):
    x = src[...]
    lo = lax.convert_element_type(
        lax.bitcast_convert_type(x[:DH, :].astype(jnp.bfloat16), jnp.uint16),
        jnp.uint32)
    hi = lax.convert_element_type(
        lax.bitcast_convert_type(x[DH:, :].astype(jnp.bfloat16), jnp.uint16),
        jnp.uint32)
    w = lax.bitcast_convert_type((hi << 16) | lo, jnp.float32)
    dst[:, pl.ds(0, DH)] = w.T
    dst[:, pl.ds(DH, DT - DH)] = jnp.zeros((TBLK, DT - DH), jnp.float32)


def _transpose_table(embT):
    return pl.pallas_call(
        _transpose_body,
        grid=(pl.cdiv(NN, TBLK),),
        in_specs=[pl.BlockSpec((D, TBLK), lambda i: (0, i))],
        out_specs=pl.BlockSpec((TBLK, DT), lambda i: (i, 0)),
        out_shape=jax.ShapeDtypeStruct((NN, DT), jnp.float32),
    )(embT)


BBLK = 2048
NBLK = B // BBLK


def _mlp1_body(se, ne, w1a, w1bl, w1bh, b1, x1, ps, psq):
    wu = lax.bitcast_convert_type(ne[...], jnp.uint32)
    ne_lo = lax.bitcast_convert_type(wu << 16, jnp.float32)
    ne_hi = lax.bitcast_convert_type(wu & jnp.uint32(0xFFFF0000), jnp.float32)
    x = jnp.dot(se[...], w1a[...], preferred_element_type=jnp.float32)
    x = x + jnp.dot(ne_lo, w1bl[...], preferred_element_type=jnp.float32)
    x = x + jnp.dot(ne_hi, w1bh[...], preferred_element_type=jnp.float32)
    x = jnp.maximum(x + b1[...], 0.0)
    x1[...] = x
    ps[...] = jnp.sum(x, axis=0, keepdims=True).reshape(1, 1, H)
    psq[...] = jnp.sum(x * x, axis=0, keepdims=True).reshape(1, 1, H)


def _mlp2_body(x1, ps, psq, g1, be1, w2, b2, y, ps2, psq2):
    m = jnp.sum(ps[...].reshape(NBLK, H), axis=0, keepdims=True) * (1.0 / B)
    ex2 = jnp.sum(psq[...].reshape(NBLK, H), axis=0, keepdims=True) * (1.0 / B)
    inv = lax.rsqrt(ex2 - m * m + 1e-5)
    x = (x1[...] - m) * (inv * g1[...]) + be1[...]
    x = jnp.maximum(jnp.dot(x, w2[...], preferred_element_type=jnp.float32) + b2[...], 0.0)
    y[...] = x
    ps2[...] = jnp.sum(x, axis=0, keepdims=True).reshape(1, 1, H)
    psq2[...] = jnp.sum(x * x, axis=0, keepdims=True).reshape(1, 1, H)


def _mlp3_body(y, ps2, psq2, g2, be2, w3, b3, w4r, b4, out):
    m = jnp.sum(ps2[...].reshape(NBLK, H), axis=0, keepdims=True) * (1.0 / B)
    ex2 = jnp.sum(psq2[...].reshape(NBLK, H), axis=0, keepdims=True) * (1.0 / B)
    inv = lax.rsqrt(ex2 - m * m + 1e-5)
    x = (y[...] - m) * (inv * g2[...]) + be2[...]
    x = jnp.maximum(jnp.dot(x, w3[...], preferred_element_type=jnp.float32) + b3[...], 0.0)
    o = jnp.sum(x * w4r[...], axis=1, keepdims=True) + b4[...]
    out[...] = 1.0 / (1.0 + jnp.exp(-o))


def _row(shape):
    return pl.BlockSpec(shape, lambda i: (0, 0))


def _blk(shape):
    return pl.BlockSpec(shape, lambda i: (i, 0))


_PSUM_OUT = pl.BlockSpec((1, 1, H), lambda i: (i, 0, 0))
_PSUM_IN = pl.BlockSpec((NBLK, 1, H), lambda i: (0, 0, 0))


def _mlp(se, ne, W1, b1, g1, be1, W2, b2, g2, be2, W3, b3, W4, b4):
    f32 = jnp.float32
    w1s = W1[:D] * (1.0 / HIST)
    w1a = (jnp.zeros((PW, H), f32)
           .at[:DH].set(w1s[:DH])
           .at[DH + 12:DH + 12 + DH].set(w1s[DH:]))
    w1bl = jnp.zeros((DT, H), f32).at[:DH].set(W1[D:D + DH])
    w1bh = jnp.zeros((DT, H), f32).at[:DH].set(W1[D + DH:])
    b1r = b1.reshape(1, H)
    g1r = g1.reshape(1, H)
    be1r = be1.reshape(1, H)
    b2r = b2.reshape(1, H)
    g2r = g2.reshape(1, H)
    be2r = be2.reshape(1, H)
    b3r = b3.reshape(1, H)
    w4r = W4.reshape(1, H)
    b4r = b4.reshape(1, 1)

    x1, ps, psq = pl.pallas_call(
        _mlp1_body,
        grid=(NBLK,),
        in_specs=[_blk((BBLK, PW)), _blk((BBLK, DT)), _row((PW, H)),
                  _row((DT, H)), _row((DT, H)), _row((1, H))],
        out_specs=[_blk((BBLK, H)), _PSUM_OUT, _PSUM_OUT],
        out_shape=[jax.ShapeDtypeStruct((B, H), f32),
                   jax.ShapeDtypeStruct((NBLK, 1, H), f32),
                   jax.ShapeDtypeStruct((NBLK, 1, H), f32)],
    )(se, ne, w1a, w1bl, w1bh, b1r)

    y, ps2, psq2 = pl.pallas_call(
        _mlp2_body,
        grid=(NBLK,),
        in_specs=[_blk((BBLK, H)), _PSUM_IN, _PSUM_IN,
                  _row((1, H)), _row((1, H)), _row((H, H)), _row((1, H))],
        out_specs=[_blk((BBLK, H)), _PSUM_OUT, _PSUM_OUT],
        out_shape=[jax.ShapeDtypeStruct((B, H), f32),
                   jax.ShapeDtypeStruct((NBLK, 1, H), f32),
                   jax.ShapeDtypeStruct((NBLK, 1, H), f32)],
    )(x1, ps, psq, g1r, be1r, W2, b2r)

    out = pl.pallas_call(
        _mlp3_body,
        grid=(NBLK,),
        in_specs=[_blk((BBLK, H)), _PSUM_IN, _PSUM_IN,
                  _row((1, H)), _row((1, H)), _row((H, H)), _row((1, H)),
                  _row((1, H)), _row((1, 1))],
        out_specs=_blk((BBLK, 1)),
        out_shape=jax.ShapeDtypeStruct((B, 1), f32),
    )(y, ps2, psq2, g2r, be2r, W3, b3r, w4r, b4r)

    return out.reshape(B)


def kernel(source, node, emb, W1, b1, g1, be1, W2, b2, g2, be2, W3, b3, W4, b4):
    emb_rm = _transpose_table(jnp.transpose(emb))
    se, ne = _sc_gather_mean(source, node, emb_rm)
    return _mlp(se, ne, W1, b1, g1, be1, W2, b2, g2, be2, W3, b3, W4, b4)


# speedup vs baseline: 3.8668x; 1.0254x over previous
import jax
import jax.numpy as jnp
from jax import lax
from jax.experimental import pallas as pl
from jax.experimental.pallas import tpu as pltpu
from jax.experimental.pallas import tpu_sc as plsc

B = 16384
HIST = 50
D = 200
DH = D // 2
DT = 128
PW = 224
H = 256
NN = 1000000
NW = 32
BPW = B // NW
OCH = 32
NCH = 128
LANES = 16
NCHUNK = 7

RPG = 2
NSTEP = BPW // RPG
NBUF = 4


def _sc_body(src_hbm, node_hbm, emb_hbm, se_hbm, ne_hbm,
             idx_all, buf0, buf1, buf2, buf3, out_v, idxn, nbuf,
             sem0, sem1, sem2, sem3, semn):
    wid = lax.axis_index("s") * 2 + lax.axis_index("c")
    wbase = pl.multiple_of(wid * BPW, BPW)

    pltpu.sync_copy(src_hbm.at[pl.ds(wid * NSTEP, NSTEP), :], idx_all)

    pltpu.sync_copy(node_hbm.at[pl.ds(wbase, BPW)], idxn)

    def node_chunk(g, c):
        off = pl.multiple_of(g * NCH, NCH)
        pltpu.async_copy(emb_hbm.at[idxn.at[pl.ds(off, NCH)]], nbuf, semn).wait()
        pltpu.sync_copy(nbuf, ne_hbm.at[pl.ds(pl.multiple_of(wbase + off, NCH), NCH), :])
        return c

    lax.fori_loop(0, BPW // NCH, node_chunk, 0)

    bufs = (buf0, buf1, buf2, buf3)
    sems = (sem0, sem1, sem2, sem3)

    def start(t, k):
        pltpu.async_copy(emb_hbm.at[idx_all.at[t]], bufs[k], sems[k])

    def wait(t, k):
        pltpu.make_async_copy(emb_hbm.at[idx_all.at[t]], bufs[k], sems[k]).wait()

    def reduce_rows(t, k):
        buf = bufs[k]
        r = lax.rem(t * RPG, OCH)
        for p in range(RPG):
            def red(j, accs, p=p):
                new = list(accs)
                for c in range(NCHUNK):
                    w = buf[p * HIST + j, pl.ds(c * LANES, LANES)]
                    lo, hi = plsc.unpack(
                        plsc.bitcast(w, jnp.bfloat16),
                        format=plsc.PackFormat.INTERLEAVED,
                    )
                    new[c] = new[c] + lo
                    new[NCHUNK + c] = new[NCHUNK + c] + hi
                return tuple(new)

            accs = lax.fori_loop(
                0, HIST, red,
                tuple(jnp.zeros((LANES,), jnp.float32) for _ in range(2 * NCHUNK)),
            )
            for c in range(2 * NCHUNK):
                out_v[r + p, pl.ds(c * LANES, LANES)] = accs[c]

    for k in range(NBUF):
        start(k, k)

    def step_k(k):
        def go(t, c):
            wait(t, k)
            reduce_rows(t, k)

            @pl.when(t + NBUF < NSTEP)
            def _():
                start(t + NBUF, k)

            @pl.when(lax.rem(t * RPG, OCH) == OCH - RPG)
            def _():
                off = pl.multiple_of(wbase + t * RPG - (OCH - RPG), OCH)
                pltpu.sync_copy(out_v, se_hbm.at[pl.ds(off, OCH), :])

            return c
        return go

    def steps(g, c):
        t0 = g * NBUF
        for k in range(NBUF):
            c = step_k(k)(t0 + k, c)
        return c

    main = (NSTEP // NBUF) * NBUF
    lax.fori_loop(0, NSTEP // NBUF, steps, 0)
    for k in range(NSTEP - main):
        step_k(k)(main + k, 0)


def _sc_gather_mean(source, node, emb):
    mesh = plsc.VectorSubcoreMesh(core_axis_name="c", subcore_axis_name="s")
    f32 = jnp.float32
    run = pl.kernel(
        _sc_body,
        out_type=(
            jax.ShapeDtypeStruct((B, PW), f32),
            jax.ShapeDtypeStruct((B, DT), f32),
        ),
        mesh=mesh,
        scratch_types=[
            pltpu.VMEM((NSTEP, RPG * HIST), jnp.int32),
            pltpu.VMEM((RPG * HIST, DT), f32),
            pltpu.VMEM((RPG * HIST, DT), f32),
            pltpu.VMEM((RPG * HIST, DT), f32),
            pltpu.VMEM((RPG * HIST, DT), f32),
            pltpu.VMEM((OCH, PW), f32),
            pltpu.VMEM((BPW,), jnp.int32),
            pltpu.VMEM((NCH, DT), f32),
            pltpu.SemaphoreType.DMA,
            pltpu.SemaphoreType.DMA,
            pltpu.SemaphoreType.DMA,
            pltpu.SemaphoreType.DMA,
            pltpu.SemaphoreType.DMA,
        ],
        compiler_params=pltpu.CompilerParams(needs_layout_passes=False),
    )
    return run(source.reshape(B // RPG, RPG * HIST), node, emb)


TBLK = 8192


def _transpose_body(src, dst):
    x = src[...]
    lo = lax.convert_element_type(
        lax.bitcast_convert_type(x[:DH, :].astype(jnp.bfloat16), jnp.uint16),
        jnp.uint32)
    hi = lax.convert_element_type(
        lax.bitcast_convert_type(x[DH:, :].astype(jnp.bfloat16), jnp.uint16),
        jnp.uint32)
    w = lax.bitcast_convert_type((hi << 16) | lo, jnp.float32)
    dst[:, pl.ds(0, DH)] = w.T
    dst[:, pl.ds(DH, DT - DH)] = jnp.zeros((TBLK, DT - DH), jnp.float32)


def _transpose_table(embT):
    return pl.pallas_call(
        _transpose_body,
        grid=(pl.cdiv(NN, TBLK),),
        in_specs=[pl.BlockSpec((D, TBLK), lambda i: (0, i))],
        out_specs=pl.BlockSpec((TBLK, DT), lambda i: (i, 0)),
        out_shape=jax.ShapeDtypeStruct((NN, DT), jnp.float32),
    )(embT)


BBLK = 2048
NBLK = B // BBLK


def _mlp1_body(se, ne, w1a, w1bl, w1bh, b1, x1, ps, psq):
    wu = lax.bitcast_convert_type(ne[...], jnp.uint32)
    ne_lo = lax.bitcast_convert_type(wu << 16, jnp.float32)
    ne_hi = lax.bitcast_convert_type(wu & jnp.uint32(0xFFFF0000), jnp.float32)
    x = jnp.dot(se[...], w1a[...], preferred_element_type=jnp.float32)
    x = x + jnp.dot(ne_lo, w1bl[...], preferred_element_type=jnp.float32)
    x = x + jnp.dot(ne_hi, w1bh[...], preferred_element_type=jnp.float32)
    x = jnp.maximum(x + b1[...], 0.0)
    x1[...] = x
    ps[...] = jnp.sum(x, axis=0, keepdims=True).reshape(1, 1, H)
    psq[...] = jnp.sum(x * x, axis=0, keepdims=True).reshape(1, 1, H)


def _mlp2_body(x1, ps, psq, g1, be1, w2, b2, y, ps2, psq2):
    m = jnp.sum(ps[...].reshape(NBLK, H), axis=0, keepdims=True) * (1.0 / B)
    ex2 = jnp.sum(psq[...].reshape(NBLK, H), axis=0, keepdims=True) * (1.0 / B)
    inv = lax.rsqrt(ex2 - m * m + 1e-5)
    x = (x1[...] - m) * (inv * g1[...]) + be1[...]
    x = jnp.maximum(jnp.dot(x, w2[...], preferred_element_type=jnp.float32) + b2[...], 0.0)
    y[...] = x
    ps2[...] = jnp.sum(x, axis=0, keepdims=True).reshape(1, 1, H)
    psq2[...] = jnp.sum(x * x, axis=0, keepdims=True).reshape(1, 1, H)


def _mlp3_body(y, ps2, psq2, g2, be2, w3, b3, w4r, b4, out):
    m = jnp.sum(ps2[...].reshape(NBLK, H), axis=0, keepdims=True) * (1.0 / B)
    ex2 = jnp.sum(psq2[...].reshape(NBLK, H), axis=0, keepdims=True) * (1.0 / B)
    inv = lax.rsqrt(ex2 - m * m + 1e-5)
    x = (y[...] - m) * (inv * g2[...]) + be2[...]
    x = jnp.maximum(jnp.dot(x, w3[...], preferred_element_type=jnp.float32) + b3[...], 0.0)
    o = jnp.sum(x * w4r[...], axis=1, keepdims=True) + b4[...]
    out[...] = 1.0 / (1.0 + jnp.exp(-o))


def _row(shape):
    return pl.BlockSpec(shape, lambda i: (0, 0))


def _blk(shape):
    return pl.BlockSpec(shape, lambda i: (i, 0))


_PSUM_OUT = pl.BlockSpec((1, 1, H), lambda i: (i, 0, 0))
_PSUM_IN = pl.BlockSpec((NBLK, 1, H), lambda i: (0, 0, 0))


def _mlp(se, ne, W1, b1, g1, be1, W2, b2, g2, be2, W3, b3, W4, b4):
    f32 = jnp.float32
    w1s = W1[:D] * (1.0 / HIST)
    w1a = (jnp.zeros((PW, H), f32)
           .at[:DH].set(w1s[:DH])
           .at[DH + 12:DH + 12 + DH].set(w1s[DH:]))
    w1bl = jnp.zeros((DT, H), f32).at[:DH].set(W1[D:D + DH])
    w1bh = jnp.zeros((DT, H), f32).at[:DH].set(W1[D + DH:])
    b1r = b1.reshape(1, H)
    g1r = g1.reshape(1, H)
    be1r = be1.reshape(1, H)
    b2r = b2.reshape(1, H)
    g2r = g2.reshape(1, H)
    be2r = be2.reshape(1, H)
    b3r = b3.reshape(1, H)
    w4r = W4.reshape(1, H)
    b4r = b4.reshape(1, 1)

    x1, ps, psq = pl.pallas_call(
        _mlp1_body,
        grid=(NBLK,),
        in_specs=[_blk((BBLK, PW)), _blk((BBLK, DT)), _row((PW, H)),
                  _row((DT, H)), _row((DT, H)), _row((1, H))],
        out_specs=[_blk((BBLK, H)), _PSUM_OUT, _PSUM_OUT],
        out_shape=[jax.ShapeDtypeStruct((B, H), f32),
                   jax.ShapeDtypeStruct((NBLK, 1, H), f32),
                   jax.ShapeDtypeStruct((NBLK, 1, H), f32)],
    )(se, ne, w1a, w1bl, w1bh, b1r)

    y, ps2, psq2 = pl.pallas_call(
        _mlp2_body,
        grid=(NBLK,),
        in_specs=[_blk((BBLK, H)), _PSUM_IN, _PSUM_IN,
                  _row((1, H)), _row((1, H)), _row((H, H)), _row((1, H))],
        out_specs=[_blk((BBLK, H)), _PSUM_OUT, _PSUM_OUT],
        out_shape=[jax.ShapeDtypeStruct((B, H), f32),
                   jax.ShapeDtypeStruct((NBLK, 1, H), f32),
                   jax.ShapeDtypeStruct((NBLK, 1, H), f32)],
    )(x1, ps, psq, g1r, be1r, W2, b2r)

    out = pl.pallas_call(
        _mlp3_body,
        grid=(NBLK,),
        in_specs=[_blk((BBLK, H)), _PSUM_IN, _PSUM_IN,
                  _row((1, H)), _row((1, H)), _row((H, H)), _row((1, H)),
                  _row((1, H)), _row((1, 1))],
        out_specs=_blk((BBLK, 1)),
        out_shape=jax.ShapeDtypeStruct((B, 1), f32),
    )(y, ps2, psq2, g2r, be2r, W3, b3r, w4r, b4r)

    return out.reshape(B)


def kernel(source, node, emb, W1, b1, g1, be1, W2, b2, g2, be2, W3, b3, W4, b4):
    emb_rm = _transpose_table(jnp.transpose(emb))
    se, ne = _sc_gather_mean(source, node, emb_rm)
    return _mlp(se, ne, W1, b1, g1, be1, W2, b2, g2, be2, W3, b3, W4, b4)


# speedup vs baseline: 3.9513x; 1.0219x over previous
import jax
import jax.numpy as jnp
from jax import lax
from jax.experimental import pallas as pl
from jax.experimental.pallas import tpu as pltpu
from jax.experimental.pallas import tpu_sc as plsc

B = 16384
HIST = 50
D = 200
DH = D // 2
DT = 128
PW = 224
H = 256
NN = 1000000
NW = 32
BPW = B // NW
OCH = 32
NCH = 128
LANES = 16
NCHUNK = 7

RPG = 2
NSTEP = BPW // RPG
NBUF = 4


def _sc_body(src_hbm, node_hbm, emb_hbm, se_hbm, ne_hbm,
             idx_all, buf0, buf1, buf2, buf3, out_v, idxn, nbuf,
             sem0, sem1, sem2, sem3, semn):
    wid = lax.axis_index("s") * 2 + lax.axis_index("c")
    wbase = pl.multiple_of(wid * BPW, BPW)

    pltpu.sync_copy(src_hbm.at[pl.ds(wid * NSTEP, NSTEP), :], idx_all)

    pltpu.sync_copy(node_hbm.at[pl.ds(wbase, BPW)], idxn)

    def node_chunk(g, c):
        off = pl.multiple_of(g * NCH, NCH)
        pltpu.async_copy(emb_hbm.at[idxn.at[pl.ds(off, NCH)]], nbuf, semn).wait()
        pltpu.sync_copy(nbuf, ne_hbm.at[pl.ds(pl.multiple_of(wbase + off, NCH), NCH), :])
        return c

    lax.fori_loop(0, BPW // NCH, node_chunk, 0)

    bufs = (buf0, buf1, buf2, buf3)
    sems = (sem0, sem1, sem2, sem3)

    def start(t, k):
        pltpu.async_copy(emb_hbm.at[idx_all.at[t]], bufs[k], sems[k])

    def wait(t, k):
        pltpu.make_async_copy(emb_hbm.at[idx_all.at[t]], bufs[k], sems[k]).wait()

    def reduce_rows(t, k):
        buf = bufs[k]
        r = lax.rem(t * RPG, OCH)
        for p in range(RPG):
            def red(j, accs, p=p):
                new = list(accs)
                for c in range(NCHUNK):
                    w = buf[p * HIST + j, pl.ds(c * LANES, LANES)]
                    lo, hi = plsc.unpack(
                        plsc.bitcast(w, jnp.bfloat16),
                        format=plsc.PackFormat.INTERLEAVED,
                    )
                    new[c] = new[c] + lo
                    new[NCHUNK + c] = new[NCHUNK + c] + hi
                return tuple(new)

            accs = lax.fori_loop(
                0, HIST, red,
                tuple(jnp.zeros((LANES,), jnp.float32) for _ in range(2 * NCHUNK)),
            )
            for c in range(2 * NCHUNK):
                out_v[r + p, pl.ds(c * LANES, LANES)] = accs[c]

    for k in range(NBUF):
        start(k, k)

    def step_k(k):
        def go(t, c):
            wait(t, k)
            reduce_rows(t, k)

            @pl.when(t + NBUF < NSTEP)
            def _():
                start(t + NBUF, k)

            @pl.when(lax.rem(t * RPG, OCH) == OCH - RPG)
            def _():
                off = pl.multiple_of(wbase + t * RPG - (OCH - RPG), OCH)
                pltpu.sync_copy(out_v, se_hbm.at[pl.ds(off, OCH), :])

            return c
        return go

    def steps(g, c):
        t0 = g * NBUF
        for k in range(NBUF):
            c = step_k(k)(t0 + k, c)
        return c

    main = (NSTEP // NBUF) * NBUF
    lax.fori_loop(0, NSTEP // NBUF, steps, 0)
    for k in range(NSTEP - main):
        step_k(k)(main + k, 0)


def _sc_gather_mean(source, node, emb):
    mesh = plsc.VectorSubcoreMesh(core_axis_name="c", subcore_axis_name="s")
    f32 = jnp.float32
    run = pl.kernel(
        _sc_body,
        out_type=(
            jax.ShapeDtypeStruct((B, PW), f32),
            jax.ShapeDtypeStruct((B, DT), f32),
        ),
        mesh=mesh,
        scratch_types=[
            pltpu.VMEM((NSTEP, RPG * HIST), jnp.int32),
            pltpu.VMEM((RPG * HIST, DT), f32),
            pltpu.VMEM((RPG * HIST, DT), f32),
            pltpu.VMEM((RPG * HIST, DT), f32),
            pltpu.VMEM((RPG * HIST, DT), f32),
            pltpu.VMEM((OCH, PW), f32),
            pltpu.VMEM((BPW,), jnp.int32),
            pltpu.VMEM((NCH, DT), f32),
            pltpu.SemaphoreType.DMA,
            pltpu.SemaphoreType.DMA,
            pltpu.SemaphoreType.DMA,
            pltpu.SemaphoreType.DMA,
            pltpu.SemaphoreType.DMA,
        ],
        compiler_params=pltpu.CompilerParams(needs_layout_passes=False),
    )
    return run(source.reshape(B // RPG, RPG * HIST), node, emb)


TBLK = 16384


def _transpose_body(src, dst):
    x = src[...]
    lo = lax.convert_element_type(
        lax.bitcast_convert_type(x[:DH, :].astype(jnp.bfloat16), jnp.uint16),
        jnp.uint32)
    hi = lax.convert_element_type(
        lax.bitcast_convert_type(x[DH:, :].astype(jnp.bfloat16), jnp.uint16),
        jnp.uint32)
    w = lax.bitcast_convert_type((hi << 16) | lo, jnp.float32)
    dst[:, pl.ds(0, DH)] = w.T
    dst[:, pl.ds(DH, DT - DH)] = jnp.zeros((TBLK, DT - DH), jnp.float32)


def _transpose_table(embT):
    return pl.pallas_call(
        _transpose_body,
        grid=(pl.cdiv(NN, TBLK),),
        in_specs=[pl.BlockSpec((D, TBLK), lambda i: (0, i))],
        out_specs=pl.BlockSpec((TBLK, DT), lambda i: (i, 0)),
        out_shape=jax.ShapeDtypeStruct((NN, DT), jnp.float32),
    )(embT)


BBLK = 2048
NBLK = B // BBLK


def _mlp1_body(se, ne, w1a, w1bl, w1bh, b1, x1, ps, psq):
    wu = lax.bitcast_convert_type(ne[...], jnp.uint32)
    ne_lo = lax.bitcast_convert_type(wu << 16, jnp.float32)
    ne_hi = lax.bitcast_convert_type(wu & jnp.uint32(0xFFFF0000), jnp.float32)
    x = jnp.dot(se[...], w1a[...], preferred_element_type=jnp.float32)
    x = x + jnp.dot(ne_lo, w1bl[...], preferred_element_type=jnp.float32)
    x = x + jnp.dot(ne_hi, w1bh[...], preferred_element_type=jnp.float32)
    x = jnp.maximum(x + b1[...], 0.0)
    x1[...] = x
    ps[...] = jnp.sum(x, axis=0, keepdims=True).reshape(1, 1, H)
    psq[...] = jnp.sum(x * x, axis=0, keepdims=True).reshape(1, 1, H)


def _mlp2_body(x1, ps, psq, g1, be1, w2, b2, y, ps2, psq2):
    m = jnp.sum(ps[...].reshape(NBLK, H), axis=0, keepdims=True) * (1.0 / B)
    ex2 = jnp.sum(psq[...].reshape(NBLK, H), axis=0, keepdims=True) * (1.0 / B)
    inv = lax.rsqrt(ex2 - m * m + 1e-5)
    x = (x1[...] - m) * (inv * g1[...]) + be1[...]
    x = jnp.maximum(jnp.dot(x, w2[...], preferred_element_type=jnp.float32) + b2[...], 0.0)
    y[...] = x
    ps2[...] = jnp.sum(x, axis=0, keepdims=True).reshape(1, 1, H)
    psq2[...] = jnp.sum(x * x, axis=0, keepdims=True).reshape(1, 1, H)


def _mlp3_body(y, ps2, psq2, g2, be2, w3, b3, w4r, b4, out):
    m = jnp.sum(ps2[...].reshape(NBLK, H), axis=0, keepdims=True) * (1.0 / B)
    ex2 = jnp.sum(psq2[...].reshape(NBLK, H), axis=0, keepdims=True) * (1.0 / B)
    inv = lax.rsqrt(ex2 - m * m + 1e-5)
    x = (y[...] - m) * (inv * g2[...]) + be2[...]
    x = jnp.maximum(jnp.dot(x, w3[...], preferred_element_type=jnp.float32) + b3[...], 0.0)
    o = jnp.sum(x * w4r[...], axis=1, keepdims=True) + b4[...]
    out[...] = 1.0 / (1.0 + jnp.exp(-o))


def _row(shape):
    return pl.BlockSpec(shape, lambda i: (0, 0))


def _blk(shape):
    return pl.BlockSpec(shape, lambda i: (i, 0))


_PSUM_OUT = pl.BlockSpec((1, 1, H), lambda i: (i, 0, 0))
_PSUM_IN = pl.BlockSpec((NBLK, 1, H), lambda i: (0, 0, 0))


def _mlp(se, ne, W1, b1, g1, be1, W2, b2, g2, be2, W3, b3, W4, b4):
    f32 = jnp.float32
    w1s = W1[:D] * (1.0 / HIST)
    w1a = (jnp.zeros((PW, H), f32)
           .at[:DH].set(w1s[:DH])
           .at[DH + 12:DH + 12 + DH].set(w1s[DH:]))
    w1bl = jnp.zeros((DT, H), f32).at[:DH].set(W1[D:D + DH])
    w1bh = jnp.zeros((DT, H), f32).at[:DH].set(W1[D + DH:])
    b1r = b1.reshape(1, H)
    g1r = g1.reshape(1, H)
    be1r = be1.reshape(1, H)
    b2r = b2.reshape(1, H)
    g2r = g2.reshape(1, H)
    be2r = be2.reshape(1, H)
    b3r = b3.reshape(1, H)
    w4r = W4.reshape(1, H)
    b4r = b4.reshape(1, 1)

    x1, ps, psq = pl.pallas_call(
        _mlp1_body,
        grid=(NBLK,),
        in_specs=[_blk((BBLK, PW)), _blk((BBLK, DT)), _row((PW, H)),
                  _row((DT, H)), _row((DT, H)), _row((1, H))],
        out_specs=[_blk((BBLK, H)), _PSUM_OUT, _PSUM_OUT],
        out_shape=[jax.ShapeDtypeStruct((B, H), f32),
                   jax.ShapeDtypeStruct((NBLK, 1, H), f32),
                   jax.ShapeDtypeStruct((NBLK, 1, H), f32)],
    )(se, ne, w1a, w1bl, w1bh, b1r)

    y, ps2, psq2 = pl.pallas_call(
        _mlp2_body,
        grid=(NBLK,),
        in_specs=[_blk((BBLK, H)), _PSUM_IN, _PSUM_IN,
                  _row((1, H)), _row((1, H)), _row((H, H)), _row((1, H))],
        out_specs=[_blk((BBLK, H)), _PSUM_OUT, _PSUM_OUT],
        out_shape=[jax.ShapeDtypeStruct((B, H), f32),
                   jax.ShapeDtypeStruct((NBLK, 1, H), f32),
                   jax.ShapeDtypeStruct((NBLK, 1, H), f32)],
    )(x1, ps, psq, g1r, be1r, W2, b2r)

    out = pl.pallas_call(
        _mlp3_body,
        grid=(NBLK,),
        in_specs=[_blk((BBLK, H)), _PSUM_IN, _PSUM_IN,
                  _row((1, H)), _row((1, H)), _row((H, H)), _row((1, H)),
                  _row((1, H)), _row((1, 1))],
        out_specs=_blk((BBLK, 1)),
        out_shape=jax.ShapeDtypeStruct((B, 1), f32),
    )(y, ps2, psq2, g2r, be2r, W3, b3r, w4r, b4r)

    return out.reshape(B)


def kernel(source, node, emb, W1, b1, g1, be1, W2, b2, g2, be2, W3, b3, W4, b4):
    emb_rm = _transpose_table(jnp.transpose(emb))
    se, ne = _sc_gather_mean(source, node, emb_rm)
    return _mlp(se, ne, W1, b1, g1, be1, W2, b2, g2, be2, W3, b3, W4, b4)
